# Initial kernel scaffold; baseline (speedup 1.0000x reference)
import functools

import jax
import jax.numpy as jnp
from jax import lax
from jax.experimental import pallas as pl
from jax.experimental.pallas import tpu as pltpu
from jax.experimental.pallas import tpu_sc as plsc

N = 10000
D = 128
E = 320000

NC = 2
NS = 16
NW = NC * NS
EPW = E // NW
K = 80
NCHUNK = EPW // K

ROWB = 632
ROWL = N - (NS - 1) * ROWB
ZB = 8

RB = 1000
NG = N // RB

_mesh = plsc.VectorSubcoreMesh(core_axis_name="c", subcore_axis_name="s")


def _deg_body(dst_hbm, out_hbm, zbuf_v, ones_v, didx_v, deg_sh):
    c = lax.axis_index("c")
    s = lax.axis_index("s")
    wid = c * NS + s

    def _fill(i, _):
        zbuf_v[pl.ds(i * 16, 16)] = jnp.zeros((16,), jnp.float32)
        ones_v[pl.ds(i * 16, 16)] = jnp.ones((16,), jnp.float32)
        return 0
    lax.fori_loop(0, (ROWB + 16) // 16, _fill, 0)

    @pl.when(s < NS - 1)
    def _():
        pltpu.sync_copy(zbuf_v.at[pl.ds(0, ROWB)], deg_sh.at[pl.ds(s * ROWB, ROWB)])

    @pl.when(s == NS - 1)
    def _():
        pltpu.sync_copy(zbuf_v.at[pl.ds(0, ROWL)], deg_sh.at[pl.ds(s * ROWB, ROWL)])

    plsc.subcore_barrier()

    def _edge(g, _):
        base = wid * EPW + g * K
        pltpu.sync_copy(dst_hbm.at[pl.ds(base, K)], didx_v)
        pltpu.sync_copy(ones_v.at[pl.ds(0, K)], deg_sh.at[didx_v], add=True)
        return 0
    lax.fori_loop(0, NCHUNK, _edge, 0)

    plsc.subcore_barrier()

    @pl.when(s < NS - 1)
    def _():
        pltpu.sync_copy(deg_sh.at[pl.ds(s * ROWB, ROWB)],
                        out_hbm.at[c, pl.ds(s * ROWB, ROWB)])

    @pl.when(s == NS - 1)
    def _():
        pltpu.sync_copy(deg_sh.at[pl.ds(s * ROWB, ROWL)],
                        out_hbm.at[c, pl.ds(s * ROWB, ROWL)])


_deg = pl.kernel(
    _deg_body,
    out_type=jax.ShapeDtypeStruct((NC, N), jnp.float32),
    mesh=_mesh,
    scratch_types=[
        pltpu.VMEM((ROWB + 16,), jnp.float32),
        pltpu.VMEM((ROWB + 16,), jnp.float32),
        pltpu.VMEM((K,), jnp.int32),
        pltpu.VMEM_SHARED((N,), jnp.float32),
    ],
)


def _edge_body(y_hbm, src_hbm, dst_hbm, out_hbm,
               zbuf_v, sidx_v, didx_v, rows_v, sem, acc_sh):
    c = lax.axis_index("c")
    s = lax.axis_index("s")
    wid = c * NS + s

    def _fill(i, _):
        zbuf_v[i % ZB, pl.ds((i // ZB) * 16, 16)] = jnp.zeros((16,), jnp.float32)
        return 0
    lax.fori_loop(0, ZB * (D // 16), _fill, 0)

    def _zero(i, _):
        pltpu.sync_copy(zbuf_v, acc_sh.at[pl.ds(s * ROWB + i * ZB, ZB)])
        return 0
    nz = jnp.where(s == NS - 1, ROWL // ZB, ROWB // ZB)
    lax.fori_loop(0, nz, _zero, 0)

    plsc.subcore_barrier()

    def _edge(g, _):
        base = wid * EPW + g * K
        pltpu.sync_copy(src_hbm.at[pl.ds(base, K)], sidx_v)
        pltpu.sync_copy(dst_hbm.at[pl.ds(base, K)], didx_v)
        pltpu.async_copy(y_hbm.at[sidx_v], rows_v, sem).wait()
        pltpu.sync_copy(rows_v, acc_sh.at[didx_v], add=True)
        return 0
    lax.fori_loop(0, NCHUNK, _edge, 0)

    plsc.subcore_barrier()

    @pl.when(s < NS - 1)
    def _():
        pltpu.sync_copy(acc_sh.at[pl.ds(s * ROWB, ROWB)],
                        out_hbm.at[c, pl.ds(s * ROWB, ROWB)])

    @pl.when(s == NS - 1)
    def _():
        pltpu.sync_copy(acc_sh.at[pl.ds(s * ROWB, ROWL)],
                        out_hbm.at[c, pl.ds(s * ROWB, ROWL)])


_edge_scatter = pl.kernel(
    _edge_body,
    out_type=jax.ShapeDtypeStruct((NC, N, D), jnp.float32),
    mesh=_mesh,
    scratch_types=[
        pltpu.VMEM((ZB, D), jnp.float32),
        pltpu.VMEM((K,), jnp.int32),
        pltpu.VMEM((K,), jnp.int32),
        pltpu.VMEM((K, D), jnp.float32),
        pltpu.SemaphoreType.DMA,
        pltpu.VMEM_SHARED((N, D), jnp.float32),
    ],
)


def _mm_body(x_ref, w_ref, o_ref):
    o_ref[...] = jnp.dot(x_ref[...], w_ref[...],
                         preferred_element_type=jnp.float32)


def _matmul(x, w):
    return pl.pallas_call(
        _mm_body,
        grid=(NG,),
        in_specs=[
            pl.BlockSpec((RB, D), lambda i: (i, 0)),
            pl.BlockSpec((D, D), lambda i: (0, 0)),
        ],
        out_specs=pl.BlockSpec((RB, D), lambda i: (i, 0)),
        out_shape=jax.ShapeDtypeStruct((N, D), jnp.float32),
    )(x, w)


def _dinv_of(degp_ref):
    deg = degp_ref[0, 0, 0, :] + degp_ref[1, 0, 0, :] + 1.0
    return lax.rsqrt(deg)


def _scale_body(xw_ref, degp_ref, y_ref):
    dinv = _dinv_of(degp_ref)
    y_ref[...] = xw_ref[...] * dinv[:, None]


def _scale(xw, degp4):
    return pl.pallas_call(
        _scale_body,
        grid=(NG,),
        in_specs=[
            pl.BlockSpec((RB, D), lambda i: (i, 0)),
            pl.BlockSpec((NC, 1, 1, RB), lambda i: (0, i, 0, 0)),
        ],
        out_specs=pl.BlockSpec((RB, D), lambda i: (i, 0)),
        out_shape=jax.ShapeDtypeStruct((N, D), jnp.float32),
    )(xw, degp4)


def _mid_body(acc_ref, y_ref, degp_ref, b_ref, w_ref, o_ref):
    dinv = _dinv_of(degp_ref)
    tot = acc_ref[0] + acc_ref[1] + y_ref[...]
    h = jnp.maximum(tot * dinv[:, None] + b_ref[...], 0.0)
    o_ref[...] = jnp.dot(h, w_ref[...],
                         preferred_element_type=jnp.float32) * dinv[:, None]


def _mid(acc, y, degp4, b, w):
    return pl.pallas_call(
        _mid_body,
        grid=(NG,),
        in_specs=[
            pl.BlockSpec((NC, RB, D), lambda i: (0, i, 0)),
            pl.BlockSpec((RB, D), lambda i: (i, 0)),
            pl.BlockSpec((NC, 1, 1, RB), lambda i: (0, i, 0, 0)),
            pl.BlockSpec((1, D), lambda i: (0, 0)),
            pl.BlockSpec((D, D), lambda i: (0, 0)),
        ],
        out_specs=pl.BlockSpec((RB, D), lambda i: (i, 0)),
        out_shape=jax.ShapeDtypeStruct((N, D), jnp.float32),
    )(acc, y, degp4, b, w)


def _final_body(acc_ref, y_ref, degp_ref, b_ref, o_ref):
    dinv = _dinv_of(degp_ref)
    tot = acc_ref[0] + acc_ref[1] + y_ref[...]
    o_ref[...] = tot * dinv[:, None] + b_ref[...]


def _final(acc, y, degp4, b):
    return pl.pallas_call(
        _final_body,
        grid=(NG,),
        in_specs=[
            pl.BlockSpec((NC, RB, D), lambda i: (0, i, 0)),
            pl.BlockSpec((RB, D), lambda i: (i, 0)),
            pl.BlockSpec((NC, 1, 1, RB), lambda i: (0, i, 0, 0)),
            pl.BlockSpec((1, D), lambda i: (0, 0)),
        ],
        out_specs=pl.BlockSpec((RB, D), lambda i: (i, 0)),
        out_shape=jax.ShapeDtypeStruct((N, D), jnp.float32),
    )(acc, y, degp4, b)


def kernel(x, edge_index, W1, b1, W2, b2):
    src = edge_index[0].astype(jnp.int32)
    dst = edge_index[1].astype(jnp.int32)
    xw1 = _matmul(x, W1)
    degp = _deg(dst)
    degp4 = degp.reshape(NC, NG, 1, RB)
    y1 = _scale(xw1, degp4)
    acc1 = _edge_scatter(y1, src, dst)
    y2 = _mid(acc1, y1, degp4, b1.reshape(1, D), W2)
    acc2 = _edge_scatter(y2, src, dst)
    return _final(acc2, y2, degp4, b2.reshape(1, D))


# speedup vs baseline: 13.5734x; 13.5734x over previous
import functools

import jax
import jax.numpy as jnp
from jax import lax
from jax.experimental import pallas as pl
from jax.experimental.pallas import tpu as pltpu
from jax.experimental.pallas import tpu_sc as plsc

N = 10000
D = 128
E = 320000

NC = 2
NS = 16
NW = NC * NS
EPW = E // NW
K = 80
NCHUNK = EPW // K

ROWB = 632
ROWL = N - (NS - 1) * ROWB
ZB = 8

PAD_N = 10240
ROWD = PAD_N // NS

RB = 1000
NG = N // RB

_mesh = plsc.VectorSubcoreMesh(core_axis_name="c", subcore_axis_name="s")


def _deg_body(dst_hbm, out_hbm, zbuf_v, ones_v, didx_v, deg_sh):
    c = lax.axis_index("c")
    s = lax.axis_index("s")
    wid = c * NS + s

    def _fill(i, _):
        zbuf_v[pl.ds(i * 16, 16)] = jnp.zeros((16,), jnp.float32)
        ones_v[pl.ds(i * 16, 16)] = jnp.ones((16,), jnp.float32)
        return 0
    lax.fori_loop(0, ROWD // 16, _fill, 0)

    pltpu.sync_copy(zbuf_v, deg_sh.at[pl.ds(s * ROWD, ROWD)])

    plsc.subcore_barrier()

    def _edge(g, _):
        base = wid * EPW + g * K
        pltpu.sync_copy(dst_hbm.at[pl.ds(base, K)], didx_v)
        pltpu.sync_copy(ones_v.at[pl.ds(0, K)], deg_sh.at[didx_v], add=True)
        return 0
    lax.fori_loop(0, NCHUNK, _edge, 0)

    plsc.subcore_barrier()

    pltpu.sync_copy(deg_sh.at[pl.ds(s * ROWD, ROWD)],
                    out_hbm.at[c, pl.ds(s * ROWD, ROWD)])


_deg = pl.kernel(
    _deg_body,
    out_type=jax.ShapeDtypeStruct((NC, PAD_N), jnp.float32),
    mesh=_mesh,
    scratch_types=[
        pltpu.VMEM((ROWD,), jnp.float32),
        pltpu.VMEM((ROWD,), jnp.float32),
        pltpu.VMEM((K,), jnp.int32),
        pltpu.VMEM_SHARED((PAD_N,), jnp.float32),
    ],
)


def _edge_body(y_hbm, src_hbm, dst_hbm, out_hbm,
               zbuf_v, sidx_v, didx_v, rows_v, sem, acc_sh):
    c = lax.axis_index("c")
    s = lax.axis_index("s")
    wid = c * NS + s

    def _fill(i, _):
        zbuf_v[i % ZB, pl.ds((i // ZB) * 16, 16)] = jnp.zeros((16,), jnp.float32)
        return 0
    lax.fori_loop(0, ZB * (D // 16), _fill, 0)

    def _zero(i, _):
        pltpu.sync_copy(zbuf_v, acc_sh.at[pl.ds(s * ROWB + i * ZB, ZB)])
        return 0
    nz = jnp.where(s == NS - 1, ROWL // ZB, ROWB // ZB)
    lax.fori_loop(0, nz, _zero, 0)

    plsc.subcore_barrier()

    def _edge(g, _):
        base = wid * EPW + g * K
        pltpu.sync_copy(src_hbm.at[pl.ds(base, K)], sidx_v)
        pltpu.sync_copy(dst_hbm.at[pl.ds(base, K)], didx_v)
        pltpu.async_copy(y_hbm.at[sidx_v], rows_v, sem).wait()
        pltpu.sync_copy(rows_v, acc_sh.at[didx_v], add=True)
        return 0
    lax.fori_loop(0, NCHUNK, _edge, 0)

    plsc.subcore_barrier()

    @pl.when(s < NS - 1)
    def _():
        pltpu.sync_copy(acc_sh.at[pl.ds(s * ROWB, ROWB)],
                        out_hbm.at[c, pl.ds(s * ROWB, ROWB)])

    @pl.when(s == NS - 1)
    def _():
        pltpu.sync_copy(acc_sh.at[pl.ds(s * ROWB, ROWL)],
                        out_hbm.at[c, pl.ds(s * ROWB, ROWL)])


_edge_scatter = pl.kernel(
    _edge_body,
    out_type=jax.ShapeDtypeStruct((NC, N, D), jnp.float32),
    mesh=_mesh,
    scratch_types=[
        pltpu.VMEM((ZB, D), jnp.float32),
        pltpu.VMEM((K,), jnp.int32),
        pltpu.VMEM((K,), jnp.int32),
        pltpu.VMEM((K, D), jnp.float32),
        pltpu.SemaphoreType.DMA,
        pltpu.VMEM_SHARED((N, D), jnp.float32),
    ],
)


def _mm_body(x_ref, w_ref, o_ref):
    o_ref[...] = jnp.dot(x_ref[...], w_ref[...],
                         preferred_element_type=jnp.float32)


def _matmul(x, w):
    return pl.pallas_call(
        _mm_body,
        grid=(NG,),
        in_specs=[
            pl.BlockSpec((RB, D), lambda i: (i, 0)),
            pl.BlockSpec((D, D), lambda i: (0, 0)),
        ],
        out_specs=pl.BlockSpec((RB, D), lambda i: (i, 0)),
        out_shape=jax.ShapeDtypeStruct((N, D), jnp.float32),
    )(x, w)


def _dinv_of(degp_ref):
    deg = degp_ref[0, 0, 0, :] + degp_ref[1, 0, 0, :] + 1.0
    return lax.rsqrt(deg)


def _scale_body(xw_ref, degp_ref, y_ref):
    dinv = _dinv_of(degp_ref)
    y_ref[...] = xw_ref[...] * dinv[:, None]


def _scale(xw, degp4):
    return pl.pallas_call(
        _scale_body,
        grid=(NG,),
        in_specs=[
            pl.BlockSpec((RB, D), lambda i: (i, 0)),
            pl.BlockSpec((NC, 1, 1, RB), lambda i: (0, i, 0, 0)),
        ],
        out_specs=pl.BlockSpec((RB, D), lambda i: (i, 0)),
        out_shape=jax.ShapeDtypeStruct((N, D), jnp.float32),
    )(xw, degp4)


def _mid_body(acc_ref, y_ref, degp_ref, b_ref, w_ref, o_ref):
    dinv = _dinv_of(degp_ref)
    tot = acc_ref[0] + acc_ref[1] + y_ref[...]
    h = jnp.maximum(tot * dinv[:, None] + b_ref[...], 0.0)
    o_ref[...] = jnp.dot(h, w_ref[...],
                         preferred_element_type=jnp.float32) * dinv[:, None]


def _mid(acc, y, degp4, b, w):
    return pl.pallas_call(
        _mid_body,
        grid=(NG,),
        in_specs=[
            pl.BlockSpec((NC, RB, D), lambda i: (0, i, 0)),
            pl.BlockSpec((RB, D), lambda i: (i, 0)),
            pl.BlockSpec((NC, 1, 1, RB), lambda i: (0, i, 0, 0)),
            pl.BlockSpec((1, D), lambda i: (0, 0)),
            pl.BlockSpec((D, D), lambda i: (0, 0)),
        ],
        out_specs=pl.BlockSpec((RB, D), lambda i: (i, 0)),
        out_shape=jax.ShapeDtypeStruct((N, D), jnp.float32),
    )(acc, y, degp4, b, w)


def _final_body(acc_ref, y_ref, degp_ref, b_ref, o_ref):
    dinv = _dinv_of(degp_ref)
    tot = acc_ref[0] + acc_ref[1] + y_ref[...]
    o_ref[...] = tot * dinv[:, None] + b_ref[...]


def _final(acc, y, degp4, b):
    return pl.pallas_call(
        _final_body,
        grid=(NG,),
        in_specs=[
            pl.BlockSpec((NC, RB, D), lambda i: (0, i, 0)),
            pl.BlockSpec((RB, D), lambda i: (i, 0)),
            pl.BlockSpec((NC, 1, 1, RB), lambda i: (0, i, 0, 0)),
            pl.BlockSpec((1, D), lambda i: (0, 0)),
        ],
        out_specs=pl.BlockSpec((RB, D), lambda i: (i, 0)),
        out_shape=jax.ShapeDtypeStruct((N, D), jnp.float32),
    )(acc, y, degp4, b)


def kernel(x, edge_index, W1, b1, W2, b2):
    src = edge_index[0].astype(jnp.int32)
    dst = edge_index[1].astype(jnp.int32)
    xw1 = _matmul(x, W1)
    degp = _deg(dst)[:, :N]
    degp4 = degp.reshape(NC, NG, 1, RB)
    y1 = _scale(xw1, degp4)
    acc1 = _edge_scatter(y1, src, dst)
    y2 = _mid(acc1, y1, degp4, b1.reshape(1, D), W2)
    acc2 = _edge_scatter(y2, src, dst)
    return _final(acc2, y2, degp4, b2.reshape(1, D))


# speedup vs baseline: 28.0096x; 2.0636x over previous
import functools

import jax
import jax.numpy as jnp
from jax import lax
from jax.experimental import pallas as pl
from jax.experimental.pallas import tpu as pltpu
from jax.experimental.pallas import tpu_sc as plsc

N = 10000
D = 128
E = 320000

NC = 2
NS = 16
NW = NC * NS
EPW = E // NW
K = 80
NCHUNK = EPW // K

ROWB = 632
ROWL = N - (NS - 1) * ROWB
ZB = 8

PAD_N = 10240
ROWD = PAD_N // NS

RB = 1000
NG = N // RB

_mesh = plsc.VectorSubcoreMesh(core_axis_name="c", subcore_axis_name="s")


def _deg_body(dst3_hbm, out_hbm, zbuf_v, ones_v, didx_v, sem, isem, deg_sh):
    c = lax.axis_index("c")
    s = lax.axis_index("s")
    wid = c * NS + s

    def _fill(i, _):
        zbuf_v[pl.ds(i * 16, 16)] = jnp.zeros((16,), jnp.float32)
        ones_v[pl.ds(i * 16, 16)] = jnp.ones((16,), jnp.float32)
        return 0
    lax.fori_loop(0, ROWD // 16, _fill, 0)

    pltpu.sync_copy(zbuf_v, deg_sh.at[pl.ds(s * ROWD, ROWD)])

    plsc.subcore_barrier()

    LAG = 2
    pltpu.sync_copy(dst3_hbm.at[wid, 0], didx_v.at[0])

    def _edge(g, _):
        @pl.when(g + 1 < NCHUNK)
        def _():
            nxt = jnp.minimum(g + 1, NCHUNK - 1)
            pltpu.async_copy(dst3_hbm.at[wid, nxt],
                             didx_v.at[lax.rem(nxt, 3)], isem)

        @pl.when(g < NCHUNK)
        def _():
            pltpu.async_copy(ones_v.at[pl.ds(0, K)],
                             deg_sh.at[didx_v.at[lax.rem(g, 3)]],
                             sem, add=True)

        @pl.when(g >= LAG)
        def _():
            pltpu.make_async_copy(ones_v.at[pl.ds(0, K)],
                                  deg_sh.at[didx_v.at[0]], sem).wait()

        @pl.when(g + 1 < NCHUNK)
        def _():
            pltpu.make_async_copy(dst3_hbm.at[wid, 0], didx_v.at[0], isem).wait()
        return 0
    lax.fori_loop(0, NCHUNK + LAG, _edge, 0)

    plsc.subcore_barrier()

    pltpu.sync_copy(deg_sh.at[pl.ds(s * ROWD, ROWD)],
                    out_hbm.at[c, pl.ds(s * ROWD, ROWD)])


_deg = pl.kernel(
    _deg_body,
    out_type=jax.ShapeDtypeStruct((NC, PAD_N), jnp.float32),
    mesh=_mesh,
    scratch_types=[
        pltpu.VMEM((ROWD,), jnp.float32),
        pltpu.VMEM((ROWD,), jnp.float32),
        pltpu.VMEM((3, K), jnp.int32),
        pltpu.SemaphoreType.DMA,
        pltpu.SemaphoreType.DMA,
        pltpu.VMEM_SHARED((PAD_N,), jnp.float32),
    ],
)


def _edge_body(y_hbm, src3_hbm, dst3_hbm, out_hbm,
               zbuf_v, sidx_v, didx_v, rows_v, gsem, isem, acc_sh):
    c = lax.axis_index("c")
    s = lax.axis_index("s")
    wid = c * NS + s

    def _fill(i, _):
        zbuf_v[i % ZB, pl.ds((i // ZB) * 16, 16)] = jnp.zeros((16,), jnp.float32)
        return 0
    lax.fori_loop(0, ZB * (D // 16), _fill, 0)

    def _zero(i, _):
        pltpu.sync_copy(zbuf_v, acc_sh.at[pl.ds(s * ROWB + i * ZB, ZB)])
        return 0
    nz = jnp.where(s == NS - 1, ROWL // ZB, ROWB // ZB)
    lax.fori_loop(0, nz, _zero, 0)

    plsc.subcore_barrier()

    def _idx_start(chunk):
        slot = lax.rem(chunk, 3)
        pltpu.async_copy(src3_hbm.at[wid, chunk], sidx_v.at[slot], isem)
        pltpu.async_copy(dst3_hbm.at[wid, chunk], didx_v.at[slot], isem)

    def _idx_wait():
        pltpu.make_async_copy(src3_hbm.at[wid, 0], sidx_v.at[0], isem).wait()
        pltpu.make_async_copy(dst3_hbm.at[wid, 0], didx_v.at[0], isem).wait()

    def _gather_start(chunk):
        pltpu.async_copy(y_hbm.at[sidx_v.at[lax.rem(chunk, 3)]],
                         rows_v.at[jnp.bitwise_and(chunk, 1)],
                         gsem.at[jnp.bitwise_and(chunk, 1)])

    def _gather_wait(chunk):
        pltpu.make_async_copy(y_hbm.at[sidx_v.at[0]], rows_v.at[0],
                              gsem.at[jnp.bitwise_and(chunk, 1)]).wait()

    _idx_start(0)
    _idx_wait()
    _gather_start(0)
    _idx_start(1)

    def _edge(g, _):
        @pl.when(g + 1 < NCHUNK)
        def _():
            _idx_wait()
            _gather_start(jnp.minimum(g + 1, NCHUNK - 1))

        @pl.when(g + 2 < NCHUNK)
        def _():
            _idx_start(jnp.minimum(g + 2, NCHUNK - 1))

        _gather_wait(g)
        pltpu.sync_copy(rows_v.at[jnp.bitwise_and(g, 1)],
                        acc_sh.at[didx_v.at[lax.rem(g, 3)]], add=True)
        return 0
    lax.fori_loop(0, NCHUNK, _edge, 0)

    plsc.subcore_barrier()

    @pl.when(s < NS - 1)
    def _():
        pltpu.sync_copy(acc_sh.at[pl.ds(s * ROWB, ROWB)],
                        out_hbm.at[c, pl.ds(s * ROWB, ROWB)])

    @pl.when(s == NS - 1)
    def _():
        pltpu.sync_copy(acc_sh.at[pl.ds(s * ROWB, ROWL)],
                        out_hbm.at[c, pl.ds(s * ROWB, ROWL)])


_edge_scatter = pl.kernel(
    _edge_body,
    out_type=jax.ShapeDtypeStruct((NC, N, D), jnp.float32),
    mesh=_mesh,
    scratch_types=[
        pltpu.VMEM((ZB, D), jnp.float32),
        pltpu.VMEM((3, K), jnp.int32),
        pltpu.VMEM((3, K), jnp.int32),
        pltpu.VMEM((2, K, D), jnp.float32),
        pltpu.SemaphoreType.DMA((2,)),
        pltpu.SemaphoreType.DMA,
        pltpu.VMEM_SHARED((N, D), jnp.float32),
    ],
)


def _mm_body(x_ref, w_ref, o_ref):
    o_ref[...] = jnp.dot(x_ref[...], w_ref[...],
                         preferred_element_type=jnp.float32)


def _matmul(x, w):
    return pl.pallas_call(
        _mm_body,
        grid=(NG,),
        in_specs=[
            pl.BlockSpec((RB, D), lambda i: (i, 0)),
            pl.BlockSpec((D, D), lambda i: (0, 0)),
        ],
        out_specs=pl.BlockSpec((RB, D), lambda i: (i, 0)),
        out_shape=jax.ShapeDtypeStruct((N, D), jnp.float32),
    )(x, w)


def _dinv_of(degp_ref):
    deg = degp_ref[0, 0, 0, :] + degp_ref[1, 0, 0, :] + 1.0
    return lax.rsqrt(deg)


def _scale_body(xw_ref, degp_ref, y_ref):
    dinv = _dinv_of(degp_ref)
    y_ref[...] = xw_ref[...] * dinv[:, None]


def _scale(xw, degp4):
    return pl.pallas_call(
        _scale_body,
        grid=(NG,),
        in_specs=[
            pl.BlockSpec((RB, D), lambda i: (i, 0)),
            pl.BlockSpec((NC, 1, 1, RB), lambda i: (0, i, 0, 0)),
        ],
        out_specs=pl.BlockSpec((RB, D), lambda i: (i, 0)),
        out_shape=jax.ShapeDtypeStruct((N, D), jnp.float32),
    )(xw, degp4)


def _mid_body(acc_ref, y_ref, degp_ref, b_ref, w_ref, o_ref):
    dinv = _dinv_of(degp_ref)
    tot = acc_ref[0] + acc_ref[1] + y_ref[...]
    h = jnp.maximum(tot * dinv[:, None] + b_ref[...], 0.0)
    o_ref[...] = jnp.dot(h, w_ref[...],
                         preferred_element_type=jnp.float32) * dinv[:, None]


def _mid(acc, y, degp4, b, w):
    return pl.pallas_call(
        _mid_body,
        grid=(NG,),
        in_specs=[
            pl.BlockSpec((NC, RB, D), lambda i: (0, i, 0)),
            pl.BlockSpec((RB, D), lambda i: (i, 0)),
            pl.BlockSpec((NC, 1, 1, RB), lambda i: (0, i, 0, 0)),
            pl.BlockSpec((1, D), lambda i: (0, 0)),
            pl.BlockSpec((D, D), lambda i: (0, 0)),
        ],
        out_specs=pl.BlockSpec((RB, D), lambda i: (i, 0)),
        out_shape=jax.ShapeDtypeStruct((N, D), jnp.float32),
    )(acc, y, degp4, b, w)


def _final_body(acc_ref, y_ref, degp_ref, b_ref, o_ref):
    dinv = _dinv_of(degp_ref)
    tot = acc_ref[0] + acc_ref[1] + y_ref[...]
    o_ref[...] = tot * dinv[:, None] + b_ref[...]


def _final(acc, y, degp4, b):
    return pl.pallas_call(
        _final_body,
        grid=(NG,),
        in_specs=[
            pl.BlockSpec((NC, RB, D), lambda i: (0, i, 0)),
            pl.BlockSpec((RB, D), lambda i: (i, 0)),
            pl.BlockSpec((NC, 1, 1, RB), lambda i: (0, i, 0, 0)),
            pl.BlockSpec((1, D), lambda i: (0, 0)),
        ],
        out_specs=pl.BlockSpec((RB, D), lambda i: (i, 0)),
        out_shape=jax.ShapeDtypeStruct((N, D), jnp.float32),
    )(acc, y, degp4, b)


def kernel(x, edge_index, W1, b1, W2, b2):
    src3 = edge_index[0].astype(jnp.int32).reshape(NW, NCHUNK, K)
    dst3 = edge_index[1].astype(jnp.int32).reshape(NW, NCHUNK, K)
    xw1 = _matmul(x, W1)
    degp = _deg(dst3)[:, :N]
    degp4 = degp.reshape(NC, NG, 1, RB)
    y1 = _scale(xw1, degp4)
    acc1 = _edge_scatter(y1, src3, dst3)
    y2 = _mid(acc1, y1, degp4, b1.reshape(1, D), W2)
    acc2 = _edge_scatter(y2, src3, dst3)
    return _final(acc2, y2, degp4, b2.reshape(1, D))


# speedup vs baseline: 30.9548x; 1.1051x over previous
import functools

import jax
import jax.numpy as jnp
from jax import lax
from jax.experimental import pallas as pl
from jax.experimental.pallas import tpu as pltpu
from jax.experimental.pallas import tpu_sc as plsc

N = 10000
D = 128
E = 320000

NC = 2
NS = 16
NW = NC * NS
EPW = E // NW
K = 80
NCHUNK = EPW // K

ROWB = 632
ROWL = N - (NS - 1) * ROWB
ZB = 8

PAD_N = 10240
ROWD = PAD_N // NS

RB = 1000
NG = N // RB

_mesh = plsc.VectorSubcoreMesh(core_axis_name="c", subcore_axis_name="s")


def _deg_body(dst3_hbm, out_hbm, zbuf_v, ones_v, didx_v, sem, isem, deg_sh):
    c = lax.axis_index("c")
    s = lax.axis_index("s")
    wid = c * NS + s

    def _fill(i, _):
        zbuf_v[pl.ds(i * 16, 16)] = jnp.zeros((16,), jnp.float32)
        ones_v[pl.ds(i * 16, 16)] = jnp.ones((16,), jnp.float32)
        return 0
    lax.fori_loop(0, ROWD // 16, _fill, 0)

    pltpu.sync_copy(zbuf_v, deg_sh.at[pl.ds(s * ROWD, ROWD)])

    plsc.subcore_barrier()

    LAG = 2
    pltpu.sync_copy(dst3_hbm.at[wid, 0], didx_v.at[0])

    def _edge(g, _):
        @pl.when(g + 1 < NCHUNK)
        def _():
            nxt = jnp.minimum(g + 1, NCHUNK - 1)
            pltpu.async_copy(dst3_hbm.at[wid, nxt],
                             didx_v.at[lax.rem(nxt, 3)], isem)

        @pl.when(g < NCHUNK)
        def _():
            pltpu.async_copy(ones_v.at[pl.ds(0, K)],
                             deg_sh.at[didx_v.at[lax.rem(g, 3)]],
                             sem, add=True)

        @pl.when(g >= LAG)
        def _():
            pltpu.make_async_copy(ones_v.at[pl.ds(0, K)],
                                  deg_sh.at[didx_v.at[0]], sem).wait()

        @pl.when(g + 1 < NCHUNK)
        def _():
            pltpu.make_async_copy(dst3_hbm.at[wid, 0], didx_v.at[0], isem).wait()
        return 0
    lax.fori_loop(0, NCHUNK + LAG, _edge, 0)

    plsc.subcore_barrier()

    pltpu.sync_copy(deg_sh.at[pl.ds(s * ROWD, ROWD)],
                    out_hbm.at[c, pl.ds(s * ROWD, ROWD)])


_deg = pl.kernel(
    _deg_body,
    out_type=jax.ShapeDtypeStruct((NC, PAD_N), jnp.float32),
    mesh=_mesh,
    scratch_types=[
        pltpu.VMEM((ROWD,), jnp.float32),
        pltpu.VMEM((ROWD,), jnp.float32),
        pltpu.VMEM((3, K), jnp.int32),
        pltpu.SemaphoreType.DMA,
        pltpu.SemaphoreType.DMA,
        pltpu.VMEM_SHARED((PAD_N,), jnp.float32),
    ],
)


def _edge_body(y_hbm, src3_hbm, dst3_hbm, out_hbm,
               zbuf_v, sidx_v, didx_v, rows_v, gsem, isem, ssem, acc_sh):
    c = lax.axis_index("c")
    s = lax.axis_index("s")
    wid = c * NS + s

    def _fill(i, _):
        zbuf_v[i % ZB, pl.ds((i // ZB) * 16, 16)] = jnp.zeros((16,), jnp.float32)
        return 0
    lax.fori_loop(0, ZB * (D // 16), _fill, 0)

    def _zero(i, _):
        pltpu.sync_copy(zbuf_v, acc_sh.at[pl.ds(s * ROWB + i * ZB, ZB)])
        return 0
    nz = jnp.where(s == NS - 1, ROWL // ZB, ROWB // ZB)
    lax.fori_loop(0, nz, _zero, 0)

    plsc.subcore_barrier()

    def _idx_start(chunk):
        slot = jnp.bitwise_and(chunk, 3)
        pltpu.async_copy(src3_hbm.at[wid, chunk], sidx_v.at[slot], isem)
        pltpu.async_copy(dst3_hbm.at[wid, chunk], didx_v.at[slot], isem)

    def _idx_wait():
        pltpu.make_async_copy(src3_hbm.at[wid, 0], sidx_v.at[0], isem).wait()
        pltpu.make_async_copy(dst3_hbm.at[wid, 0], didx_v.at[0], isem).wait()

    def _gather_start(chunk):
        pltpu.async_copy(y_hbm.at[sidx_v.at[jnp.bitwise_and(chunk, 3)]],
                         rows_v.at[lax.rem(chunk, 3)],
                         gsem.at[jnp.bitwise_and(chunk, 1)])

    def _gather_wait(chunk):
        pltpu.make_async_copy(y_hbm.at[sidx_v.at[0]], rows_v.at[0],
                              gsem.at[jnp.bitwise_and(chunk, 1)]).wait()

    def _scatter_start(chunk):
        slot = lax.rem(chunk, 3)
        pltpu.async_copy(rows_v.at[slot],
                         acc_sh.at[didx_v.at[jnp.bitwise_and(chunk, 3)]],
                         ssem.at[slot], add=True)

    def _scatter_wait(chunk):
        pltpu.make_async_copy(y_hbm.at[sidx_v.at[0]], rows_v.at[0],
                              ssem.at[lax.rem(chunk, 3)]).wait()

    _idx_start(0)
    _idx_wait()
    _gather_start(0)
    _idx_start(1)

    def _edge(g, _):
        @pl.when(g + 1 < NCHUNK)
        def _():
            _idx_wait()

            @pl.when(g >= 2)
            def _():
                _scatter_wait(g - 2)
            _gather_start(jnp.minimum(g + 1, NCHUNK - 1))

        @pl.when(g + 2 < NCHUNK)
        def _():
            _idx_start(jnp.minimum(g + 2, NCHUNK - 1))

        _gather_wait(g)
        _scatter_start(g)
        return 0
    lax.fori_loop(0, NCHUNK, _edge, 0)

    _scatter_wait(NCHUNK - 3)
    _scatter_wait(NCHUNK - 2)
    _scatter_wait(NCHUNK - 1)

    plsc.subcore_barrier()

    @pl.when(s < NS - 1)
    def _():
        pltpu.sync_copy(acc_sh.at[pl.ds(s * ROWB, ROWB)],
                        out_hbm.at[c, pl.ds(s * ROWB, ROWB)])

    @pl.when(s == NS - 1)
    def _():
        pltpu.sync_copy(acc_sh.at[pl.ds(s * ROWB, ROWL)],
                        out_hbm.at[c, pl.ds(s * ROWB, ROWL)])


_edge_scatter = pl.kernel(
    _edge_body,
    out_type=jax.ShapeDtypeStruct((NC, N, D), jnp.float32),
    mesh=_mesh,
    scratch_types=[
        pltpu.VMEM((ZB, D), jnp.float32),
        pltpu.VMEM((4, K), jnp.int32),
        pltpu.VMEM((4, K), jnp.int32),
        pltpu.VMEM((3, K, D), jnp.float32),
        pltpu.SemaphoreType.DMA((2,)),
        pltpu.SemaphoreType.DMA,
        pltpu.SemaphoreType.DMA((3,)),
        pltpu.VMEM_SHARED((N, D), jnp.float32),
    ],
)


def _mm_body(x_ref, w_ref, o_ref):
    o_ref[...] = jnp.dot(x_ref[...], w_ref[...],
                         preferred_element_type=jnp.float32)


def _matmul(x, w):
    return pl.pallas_call(
        _mm_body,
        grid=(NG,),
        in_specs=[
            pl.BlockSpec((RB, D), lambda i: (i, 0)),
            pl.BlockSpec((D, D), lambda i: (0, 0)),
        ],
        out_specs=pl.BlockSpec((RB, D), lambda i: (i, 0)),
        out_shape=jax.ShapeDtypeStruct((N, D), jnp.float32),
    )(x, w)


def _dinv_of(degp_ref):
    deg = degp_ref[0, 0, 0, :] + degp_ref[1, 0, 0, :] + 1.0
    return lax.rsqrt(deg)


def _scale_body(xw_ref, degp_ref, y_ref):
    dinv = _dinv_of(degp_ref)
    y_ref[...] = xw_ref[...] * dinv[:, None]


def _scale(xw, degp4):
    return pl.pallas_call(
        _scale_body,
        grid=(NG,),
        in_specs=[
            pl.BlockSpec((RB, D), lambda i: (i, 0)),
            pl.BlockSpec((NC, 1, 1, RB), lambda i: (0, i, 0, 0)),
        ],
        out_specs=pl.BlockSpec((RB, D), lambda i: (i, 0)),
        out_shape=jax.ShapeDtypeStruct((N, D), jnp.float32),
    )(xw, degp4)


def _mid_body(acc_ref, y_ref, degp_ref, b_ref, w_ref, o_ref):
    dinv = _dinv_of(degp_ref)
    tot = acc_ref[0] + acc_ref[1] + y_ref[...]
    h = jnp.maximum(tot * dinv[:, None] + b_ref[...], 0.0)
    o_ref[...] = jnp.dot(h, w_ref[...],
                         preferred_element_type=jnp.float32) * dinv[:, None]


def _mid(acc, y, degp4, b, w):
    return pl.pallas_call(
        _mid_body,
        grid=(NG,),
        in_specs=[
            pl.BlockSpec((NC, RB, D), lambda i: (0, i, 0)),
            pl.BlockSpec((RB, D), lambda i: (i, 0)),
            pl.BlockSpec((NC, 1, 1, RB), lambda i: (0, i, 0, 0)),
            pl.BlockSpec((1, D), lambda i: (0, 0)),
            pl.BlockSpec((D, D), lambda i: (0, 0)),
        ],
        out_specs=pl.BlockSpec((RB, D), lambda i: (i, 0)),
        out_shape=jax.ShapeDtypeStruct((N, D), jnp.float32),
    )(acc, y, degp4, b, w)


def _final_body(acc_ref, y_ref, degp_ref, b_ref, o_ref):
    dinv = _dinv_of(degp_ref)
    tot = acc_ref[0] + acc_ref[1] + y_ref[...]
    o_ref[...] = tot * dinv[:, None] + b_ref[...]


def _final(acc, y, degp4, b):
    return pl.pallas_call(
        _final_body,
        grid=(NG,),
        in_specs=[
            pl.BlockSpec((NC, RB, D), lambda i: (0, i, 0)),
            pl.BlockSpec((RB, D), lambda i: (i, 0)),
            pl.BlockSpec((NC, 1, 1, RB), lambda i: (0, i, 0, 0)),
            pl.BlockSpec((1, D), lambda i: (0, 0)),
        ],
        out_specs=pl.BlockSpec((RB, D), lambda i: (i, 0)),
        out_shape=jax.ShapeDtypeStruct((N, D), jnp.float32),
    )(acc, y, degp4, b)


def kernel(x, edge_index, W1, b1, W2, b2):
    src3 = edge_index[0].astype(jnp.int32).reshape(NW, NCHUNK, K)
    dst3 = edge_index[1].astype(jnp.int32).reshape(NW, NCHUNK, K)
    xw1 = _matmul(x, W1)
    degp = _deg(dst3)[:, :N]
    degp4 = degp.reshape(NC, NG, 1, RB)
    y1 = _scale(xw1, degp4)
    acc1 = _edge_scatter(y1, src3, dst3)
    y2 = _mid(acc1, y1, degp4, b1.reshape(1, D), W2)
    acc2 = _edge_scatter(y2, src3, dst3)
    return _final(acc2, y2, degp4, b2.reshape(1, D))


# speedup vs baseline: 31.5428x; 1.0190x over previous
import functools

import jax
import jax.numpy as jnp
from jax import lax
from jax.experimental import pallas as pl
from jax.experimental.pallas import tpu as pltpu
from jax.experimental.pallas import tpu_sc as plsc

N = 10000
D = 128
E = 320000

NC = 2
NS = 16
NW = NC * NS
EPW = E // NW
K = 80
NCHUNK = EPW // K

ROWB = 632
ROWL = N - (NS - 1) * ROWB
ZB = 8

PAD_N = 10240
ROWD = PAD_N // NS

RB = 1000
NG = N // RB

_mesh = plsc.VectorSubcoreMesh(core_axis_name="c", subcore_axis_name="s")


def _deg_body(dst3_hbm, out_hbm, zbuf_v, ones_v, didx_v, sem, isem, deg_sh):
    c = lax.axis_index("c")
    s = lax.axis_index("s")
    wid = c * NS + s

    def _fill(i, _):
        zbuf_v[pl.ds(i * 16, 16)] = jnp.zeros((16,), jnp.float32)
        ones_v[pl.ds(i * 16, 16)] = jnp.ones((16,), jnp.float32)
        return 0
    lax.fori_loop(0, ROWD // 16, _fill, 0)

    pltpu.sync_copy(zbuf_v, deg_sh.at[pl.ds(s * ROWD, ROWD)])

    plsc.subcore_barrier()

    LAG = 2
    pltpu.sync_copy(dst3_hbm.at[wid, 0], didx_v.at[0])

    def _edge(g, _):
        @pl.when(g + 1 < NCHUNK)
        def _():
            nxt = jnp.minimum(g + 1, NCHUNK - 1)
            pltpu.async_copy(dst3_hbm.at[wid, nxt],
                             didx_v.at[lax.rem(nxt, 3)], isem)

        @pl.when(g < NCHUNK)
        def _():
            pltpu.async_copy(ones_v.at[pl.ds(0, K)],
                             deg_sh.at[didx_v.at[lax.rem(g, 3)]],
                             sem, add=True)

        @pl.when(g >= LAG)
        def _():
            pltpu.make_async_copy(ones_v.at[pl.ds(0, K)],
                                  deg_sh.at[didx_v.at[0]], sem).wait()

        @pl.when(g + 1 < NCHUNK)
        def _():
            pltpu.make_async_copy(dst3_hbm.at[wid, 0], didx_v.at[0], isem).wait()
        return 0
    lax.fori_loop(0, NCHUNK + LAG, _edge, 0)

    plsc.subcore_barrier()

    pltpu.sync_copy(deg_sh.at[pl.ds(s * ROWD, ROWD)],
                    out_hbm.at[c, pl.ds(s * ROWD, ROWD)])


_deg = pl.kernel(
    _deg_body,
    out_type=jax.ShapeDtypeStruct((NC, PAD_N), jnp.float32),
    mesh=_mesh,
    scratch_types=[
        pltpu.VMEM((ROWD,), jnp.float32),
        pltpu.VMEM((ROWD,), jnp.float32),
        pltpu.VMEM((3, K), jnp.int32),
        pltpu.SemaphoreType.DMA,
        pltpu.SemaphoreType.DMA,
        pltpu.VMEM_SHARED((PAD_N,), jnp.float32),
    ],
)


def _edge_body(y_hbm, src3_hbm, dst3_hbm, out_hbm,
               zbuf_v, sidx_v, didx_v, rows_v, gsem, isem, ssem, acc_sh):
    c = lax.axis_index("c")
    s = lax.axis_index("s")
    wid = c * NS + s

    def _fill(i, _):
        zbuf_v[i % ZB, pl.ds((i // ZB) * 16, 16)] = jnp.zeros((16,), jnp.float32)
        return 0
    lax.fori_loop(0, ZB * (D // 16), _fill, 0)

    def _zero(i, _):
        pltpu.sync_copy(zbuf_v, acc_sh.at[pl.ds(s * ROWB + i * ZB, ZB)])
        return 0
    nz = jnp.where(s == NS - 1, ROWL // ZB, ROWB // ZB)
    lax.fori_loop(0, nz, _zero, 0)

    plsc.subcore_barrier()

    def _idx_start(chunk):
        slot = jnp.bitwise_and(chunk, 7)
        pltpu.async_copy(src3_hbm.at[wid, chunk], sidx_v.at[slot], isem)
        pltpu.async_copy(dst3_hbm.at[wid, chunk], didx_v.at[slot], isem)

    def _idx_wait():
        pltpu.make_async_copy(src3_hbm.at[wid, 0], sidx_v.at[0], isem).wait()
        pltpu.make_async_copy(dst3_hbm.at[wid, 0], didx_v.at[0], isem).wait()

    def _gather_start(chunk):
        slot = jnp.bitwise_and(chunk, 3)
        pltpu.async_copy(y_hbm.at[sidx_v.at[jnp.bitwise_and(chunk, 7)]],
                         rows_v.at[slot], gsem.at[slot])

    def _gather_wait(chunk):
        pltpu.make_async_copy(y_hbm.at[sidx_v.at[0]], rows_v.at[0],
                              gsem.at[jnp.bitwise_and(chunk, 3)]).wait()

    def _scatter_start(chunk):
        slot = jnp.bitwise_and(chunk, 3)
        pltpu.async_copy(rows_v.at[slot],
                         acc_sh.at[didx_v.at[jnp.bitwise_and(chunk, 7)]],
                         ssem.at[slot], add=True)

    def _scatter_wait(chunk):
        pltpu.make_async_copy(y_hbm.at[sidx_v.at[0]], rows_v.at[0],
                              ssem.at[jnp.bitwise_and(chunk, 3)]).wait()

    _idx_start(0)
    _idx_wait()
    _gather_start(0)
    _idx_start(1)
    _idx_wait()
    _gather_start(1)
    _idx_start(2)

    def _edge(g, _):
        @pl.when(g + 2 < NCHUNK)
        def _():
            _idx_wait()

            @pl.when(g >= 2)
            def _():
                _scatter_wait(g - 2)
            _gather_start(jnp.minimum(g + 2, NCHUNK - 1))

        @pl.when(g + 3 < NCHUNK)
        def _():
            _idx_start(jnp.minimum(g + 3, NCHUNK - 1))

        _gather_wait(g)
        _scatter_start(g)
        return 0
    lax.fori_loop(0, NCHUNK, _edge, 0)

    _scatter_wait(NCHUNK - 4)
    _scatter_wait(NCHUNK - 3)
    _scatter_wait(NCHUNK - 2)
    _scatter_wait(NCHUNK - 1)

    plsc.subcore_barrier()

    @pl.when(s < NS - 1)
    def _():
        pltpu.sync_copy(acc_sh.at[pl.ds(s * ROWB, ROWB)],
                        out_hbm.at[c, pl.ds(s * ROWB, ROWB)])

    @pl.when(s == NS - 1)
    def _():
        pltpu.sync_copy(acc_sh.at[pl.ds(s * ROWB, ROWL)],
                        out_hbm.at[c, pl.ds(s * ROWB, ROWL)])


_edge_scatter = pl.kernel(
    _edge_body,
    out_type=jax.ShapeDtypeStruct((NC, N, D), jnp.float32),
    mesh=_mesh,
    scratch_types=[
        pltpu.VMEM((ZB, D), jnp.float32),
        pltpu.VMEM((8, K), jnp.int32),
        pltpu.VMEM((8, K), jnp.int32),
        pltpu.VMEM((4, K, D), jnp.float32),
        pltpu.SemaphoreType.DMA((4,)),
        pltpu.SemaphoreType.DMA,
        pltpu.SemaphoreType.DMA((4,)),
        pltpu.VMEM_SHARED((N, D), jnp.float32),
    ],
)


def _mm_body(x_ref, w_ref, o_ref):
    o_ref[...] = jnp.dot(x_ref[...], w_ref[...],
                         preferred_element_type=jnp.float32)


def _matmul(x, w):
    return pl.pallas_call(
        _mm_body,
        grid=(NG,),
        in_specs=[
            pl.BlockSpec((RB, D), lambda i: (i, 0)),
            pl.BlockSpec((D, D), lambda i: (0, 0)),
        ],
        out_specs=pl.BlockSpec((RB, D), lambda i: (i, 0)),
        out_shape=jax.ShapeDtypeStruct((N, D), jnp.float32),
    )(x, w)


def _dinv_of(degp_ref):
    deg = degp_ref[0, 0, 0, :] + degp_ref[1, 0, 0, :] + 1.0
    return lax.rsqrt(deg)


def _scale_body(xw_ref, degp_ref, y_ref):
    dinv = _dinv_of(degp_ref)
    y_ref[...] = xw_ref[...] * dinv[:, None]


def _scale(xw, degp4):
    return pl.pallas_call(
        _scale_body,
        grid=(NG,),
        in_specs=[
            pl.BlockSpec((RB, D), lambda i: (i, 0)),
            pl.BlockSpec((NC, 1, 1, RB), lambda i: (0, i, 0, 0)),
        ],
        out_specs=pl.BlockSpec((RB, D), lambda i: (i, 0)),
        out_shape=jax.ShapeDtypeStruct((N, D), jnp.float32),
    )(xw, degp4)


def _mid_body(acc_ref, y_ref, degp_ref, b_ref, w_ref, o_ref):
    dinv = _dinv_of(degp_ref)
    tot = acc_ref[0] + acc_ref[1] + y_ref[...]
    h = jnp.maximum(tot * dinv[:, None] + b_ref[...], 0.0)
    o_ref[...] = jnp.dot(h, w_ref[...],
                         preferred_element_type=jnp.float32) * dinv[:, None]


def _mid(acc, y, degp4, b, w):
    return pl.pallas_call(
        _mid_body,
        grid=(NG,),
        in_specs=[
            pl.BlockSpec((NC, RB, D), lambda i: (0, i, 0)),
            pl.BlockSpec((RB, D), lambda i: (i, 0)),
            pl.BlockSpec((NC, 1, 1, RB), lambda i: (0, i, 0, 0)),
            pl.BlockSpec((1, D), lambda i: (0, 0)),
            pl.BlockSpec((D, D), lambda i: (0, 0)),
        ],
        out_specs=pl.BlockSpec((RB, D), lambda i: (i, 0)),
        out_shape=jax.ShapeDtypeStruct((N, D), jnp.float32),
    )(acc, y, degp4, b, w)


def _final_body(acc_ref, y_ref, degp_ref, b_ref, o_ref):
    dinv = _dinv_of(degp_ref)
    tot = acc_ref[0] + acc_ref[1] + y_ref[...]
    o_ref[...] = tot * dinv[:, None] + b_ref[...]


def _final(acc, y, degp4, b):
    return pl.pallas_call(
        _final_body,
        grid=(NG,),
        in_specs=[
            pl.BlockSpec((NC, RB, D), lambda i: (0, i, 0)),
            pl.BlockSpec((RB, D), lambda i: (i, 0)),
            pl.BlockSpec((NC, 1, 1, RB), lambda i: (0, i, 0, 0)),
            pl.BlockSpec((1, D), lambda i: (0, 0)),
        ],
        out_specs=pl.BlockSpec((RB, D), lambda i: (i, 0)),
        out_shape=jax.ShapeDtypeStruct((N, D), jnp.float32),
    )(acc, y, degp4, b)


def kernel(x, edge_index, W1, b1, W2, b2):
    src3 = edge_index[0].astype(jnp.int32).reshape(NW, NCHUNK, K)
    dst3 = edge_index[1].astype(jnp.int32).reshape(NW, NCHUNK, K)
    xw1 = _matmul(x, W1)
    degp = _deg(dst3)[:, :N]
    degp4 = degp.reshape(NC, NG, 1, RB)
    y1 = _scale(xw1, degp4)
    acc1 = _edge_scatter(y1, src3, dst3)
    y2 = _mid(acc1, y1, degp4, b1.reshape(1, D), W2)
    acc2 = _edge_scatter(y2, src3, dst3)
    return _final(acc2, y2, degp4, b2.reshape(1, D))


# speedup vs baseline: 34.8390x; 1.1045x over previous
import functools

import jax
import jax.numpy as jnp
from jax import lax
from jax.experimental import pallas as pl
from jax.experimental.pallas import tpu as pltpu
from jax.experimental.pallas import tpu_sc as plsc

N = 10000
D = 128
E = 320000

NC = 2
NS = 16
NW = NC * NS
EPW = E // NW
K = 80
NCHUNK = EPW // K

ROWB = 632
ROWL = N - (NS - 1) * ROWB
ZB = 8

PAD_N = 10240
ROWD = PAD_N // NS

RB = 1000
NG = N // RB

_mesh = plsc.VectorSubcoreMesh(core_axis_name="c", subcore_axis_name="s")


def _deg_body(dst3_hbm, out_hbm, zbuf_v, ones_v, didx_v, sem, isem, deg_sh):
    c = lax.axis_index("c")
    s = lax.axis_index("s")
    wid = c * NS + s

    def _fill(i, _):
        zbuf_v[pl.ds(i * 16, 16)] = jnp.zeros((16,), jnp.float32)
        ones_v[pl.ds(i * 16, 16)] = jnp.ones((16,), jnp.float32)
        return 0
    lax.fori_loop(0, ROWD // 16, _fill, 0)

    pltpu.sync_copy(zbuf_v, deg_sh.at[pl.ds(s * ROWD, ROWD)])

    plsc.subcore_barrier()

    pltpu.sync_copy(dst3_hbm.at[wid, 0], didx_v.at[0])
    pltpu.async_copy(dst3_hbm.at[wid, 1], didx_v.at[1], isem)

    def _edge(g, _):
        @pl.when(g + 2 < NCHUNK)
        def _():
            nxt = jnp.minimum(g + 2, NCHUNK - 1)
            pltpu.async_copy(dst3_hbm.at[wid, nxt],
                             didx_v.at[jnp.bitwise_and(nxt, 7)], isem)

        @pl.when(g >= 6)
        def _():
            pltpu.make_async_copy(ones_v.at[pl.ds(0, K)],
                                  deg_sh.at[didx_v.at[0]],
                                  sem.at[lax.rem(g, 6)]).wait()

        pltpu.async_copy(ones_v.at[pl.ds(0, K)],
                         deg_sh.at[didx_v.at[jnp.bitwise_and(g, 7)]],
                         sem.at[lax.rem(g, 6)], add=True)

        @pl.when(g + 1 < NCHUNK)
        def _():
            pltpu.make_async_copy(dst3_hbm.at[wid, 0], didx_v.at[0], isem).wait()
        return 0
    lax.fori_loop(0, NCHUNK, _edge, 0)

    def _drain(g, _):
        pltpu.make_async_copy(ones_v.at[pl.ds(0, K)], deg_sh.at[didx_v.at[0]],
                              sem.at[lax.rem(g, 6)]).wait()
        return 0
    lax.fori_loop(NCHUNK - 6, NCHUNK, _drain, 0)

    plsc.subcore_barrier()

    pltpu.sync_copy(deg_sh.at[pl.ds(s * ROWD, ROWD)],
                    out_hbm.at[c, pl.ds(s * ROWD, ROWD)])


_deg = pl.kernel(
    _deg_body,
    out_type=jax.ShapeDtypeStruct((NC, PAD_N), jnp.float32),
    mesh=_mesh,
    scratch_types=[
        pltpu.VMEM((ROWD,), jnp.float32),
        pltpu.VMEM((ROWD,), jnp.float32),
        pltpu.VMEM((8, K), jnp.int32),
        pltpu.SemaphoreType.DMA((6,)),
        pltpu.SemaphoreType.DMA,
        pltpu.VMEM_SHARED((PAD_N,), jnp.float32),
    ],
)


def _edge_body(y_hbm, src3_hbm, dst3_hbm, out_hbm,
               zbuf_v, sidx_v, didx_v, rows_v, gsem, isem, ssem, acc_sh):
    c = lax.axis_index("c")
    s = lax.axis_index("s")
    wid = c * NS + s

    def _fill(i, _):
        zbuf_v[i % ZB, pl.ds((i // ZB) * 16, 16)] = jnp.zeros((16,), jnp.float32)
        return 0
    lax.fori_loop(0, ZB * (D // 16), _fill, 0)

    def _zero(i, _):
        pltpu.sync_copy(zbuf_v, acc_sh.at[pl.ds(s * ROWB + i * ZB, ZB)])
        return 0
    nz = jnp.where(s == NS - 1, ROWL // ZB, ROWB // ZB)
    lax.fori_loop(0, nz, _zero, 0)

    plsc.subcore_barrier()

    def _idx_start(chunk):
        slot = jnp.bitwise_and(chunk, 7)
        pltpu.async_copy(src3_hbm.at[wid, chunk], sidx_v.at[slot], isem)
        pltpu.async_copy(dst3_hbm.at[wid, chunk], didx_v.at[slot], isem)

    def _idx_wait():
        pltpu.make_async_copy(src3_hbm.at[wid, 0], sidx_v.at[0], isem).wait()
        pltpu.make_async_copy(dst3_hbm.at[wid, 0], didx_v.at[0], isem).wait()

    def _gather_start(chunk):
        slot = jnp.bitwise_and(chunk, 3)
        pltpu.async_copy(y_hbm.at[sidx_v.at[jnp.bitwise_and(chunk, 7)]],
                         rows_v.at[slot], gsem.at[slot])

    def _gather_wait(chunk):
        pltpu.make_async_copy(y_hbm.at[sidx_v.at[0]], rows_v.at[0],
                              gsem.at[jnp.bitwise_and(chunk, 3)]).wait()

    def _scatter_start(chunk):
        slot = jnp.bitwise_and(chunk, 3)
        pltpu.async_copy(rows_v.at[slot],
                         acc_sh.at[didx_v.at[jnp.bitwise_and(chunk, 7)]],
                         ssem.at[slot], add=True)

    def _scatter_wait(chunk):
        pltpu.make_async_copy(y_hbm.at[sidx_v.at[0]], rows_v.at[0],
                              ssem.at[jnp.bitwise_and(chunk, 3)]).wait()

    _idx_start(0)
    _idx_wait()
    _gather_start(0)
    _idx_start(1)
    _idx_wait()
    _gather_start(1)
    _idx_start(2)

    def _edge(g, _):
        @pl.when(g + 2 < NCHUNK)
        def _():
            _idx_wait()

            @pl.when(g >= 2)
            def _():
                _scatter_wait(g - 2)
            _gather_start(jnp.minimum(g + 2, NCHUNK - 1))

        @pl.when(g + 3 < NCHUNK)
        def _():
            _idx_start(jnp.minimum(g + 3, NCHUNK - 1))

        _gather_wait(g)
        _scatter_start(g)
        return 0
    lax.fori_loop(0, NCHUNK, _edge, 0)

    _scatter_wait(NCHUNK - 4)
    _scatter_wait(NCHUNK - 3)
    _scatter_wait(NCHUNK - 2)
    _scatter_wait(NCHUNK - 1)

    plsc.subcore_barrier()

    @pl.when(s < NS - 1)
    def _():
        pltpu.sync_copy(acc_sh.at[pl.ds(s * ROWB, ROWB)],
                        out_hbm.at[c, pl.ds(s * ROWB, ROWB)])

    @pl.when(s == NS - 1)
    def _():
        pltpu.sync_copy(acc_sh.at[pl.ds(s * ROWB, ROWL)],
                        out_hbm.at[c, pl.ds(s * ROWB, ROWL)])


_edge_scatter = pl.kernel(
    _edge_body,
    out_type=jax.ShapeDtypeStruct((NC, N, D), jnp.float32),
    mesh=_mesh,
    scratch_types=[
        pltpu.VMEM((ZB, D), jnp.float32),
        pltpu.VMEM((8, K), jnp.int32),
        pltpu.VMEM((8, K), jnp.int32),
        pltpu.VMEM((4, K, D), jnp.float32),
        pltpu.SemaphoreType.DMA((4,)),
        pltpu.SemaphoreType.DMA,
        pltpu.SemaphoreType.DMA((4,)),
        pltpu.VMEM_SHARED((N, D), jnp.float32),
    ],
)


def _dinv_of(degp_ref):
    deg = degp_ref[0, 0, 0, :] + degp_ref[1, 0, 0, :] + 1.0
    return lax.rsqrt(deg)


def _mmscale_body(x_ref, w_ref, degp_ref, y_ref):
    dinv = _dinv_of(degp_ref)
    y_ref[...] = jnp.dot(x_ref[...], w_ref[...],
                         preferred_element_type=jnp.float32) * dinv[:, None]


def _mmscale(x, w, degp4):
    return pl.pallas_call(
        _mmscale_body,
        grid=(NG,),
        in_specs=[
            pl.BlockSpec((RB, D), lambda i: (i, 0)),
            pl.BlockSpec((D, D), lambda i: (0, 0)),
            pl.BlockSpec((NC, 1, 1, RB), lambda i: (0, i, 0, 0)),
        ],
        out_specs=pl.BlockSpec((RB, D), lambda i: (i, 0)),
        out_shape=jax.ShapeDtypeStruct((N, D), jnp.float32),
    )(x, w, degp4)


def _mid_body(acc_ref, y_ref, degp_ref, b_ref, w_ref, o_ref):
    dinv = _dinv_of(degp_ref)
    tot = acc_ref[0] + acc_ref[1] + y_ref[...]
    h = jnp.maximum(tot * dinv[:, None] + b_ref[...], 0.0)
    o_ref[...] = jnp.dot(h, w_ref[...],
                         preferred_element_type=jnp.float32) * dinv[:, None]


def _mid(acc, y, degp4, b, w):
    return pl.pallas_call(
        _mid_body,
        grid=(NG,),
        in_specs=[
            pl.BlockSpec((NC, RB, D), lambda i: (0, i, 0)),
            pl.BlockSpec((RB, D), lambda i: (i, 0)),
            pl.BlockSpec((NC, 1, 1, RB), lambda i: (0, i, 0, 0)),
            pl.BlockSpec((1, D), lambda i: (0, 0)),
            pl.BlockSpec((D, D), lambda i: (0, 0)),
        ],
        out_specs=pl.BlockSpec((RB, D), lambda i: (i, 0)),
        out_shape=jax.ShapeDtypeStruct((N, D), jnp.float32),
    )(acc, y, degp4, b, w)


def _final_body(acc_ref, y_ref, degp_ref, b_ref, o_ref):
    dinv = _dinv_of(degp_ref)
    tot = acc_ref[0] + acc_ref[1] + y_ref[...]
    o_ref[...] = tot * dinv[:, None] + b_ref[...]


def _final(acc, y, degp4, b):
    return pl.pallas_call(
        _final_body,
        grid=(NG,),
        in_specs=[
            pl.BlockSpec((NC, RB, D), lambda i: (0, i, 0)),
            pl.BlockSpec((RB, D), lambda i: (i, 0)),
            pl.BlockSpec((NC, 1, 1, RB), lambda i: (0, i, 0, 0)),
            pl.BlockSpec((1, D), lambda i: (0, 0)),
        ],
        out_specs=pl.BlockSpec((RB, D), lambda i: (i, 0)),
        out_shape=jax.ShapeDtypeStruct((N, D), jnp.float32),
    )(acc, y, degp4, b)


def kernel(x, edge_index, W1, b1, W2, b2):
    src3 = edge_index[0].astype(jnp.int32).reshape(NW, NCHUNK, K)
    dst3 = edge_index[1].astype(jnp.int32).reshape(NW, NCHUNK, K)
    degp = _deg(dst3)[:, :N]
    degp4 = degp.reshape(NC, NG, 1, RB)
    y1 = _mmscale(x, W1, degp4)
    acc1 = _edge_scatter(y1, src3, dst3)
    y2 = _mid(acc1, y1, degp4, b1.reshape(1, D), W2)
    acc2 = _edge_scatter(y2, src3, dst3)
    return _final(acc2, y2, degp4, b2.reshape(1, D))


# speedup vs baseline: 35.3159x; 1.0137x over previous
import functools

import jax
import jax.numpy as jnp
from jax import lax
from jax.experimental import pallas as pl
from jax.experimental.pallas import tpu as pltpu
from jax.experimental.pallas import tpu_sc as plsc

N = 10000
D = 128
E = 320000

NC = 2
NS = 16
NW = NC * NS
EPW = E // NW
K = 80
NCHUNK = EPW // K

ROWB = 632
ROWL = N - (NS - 1) * ROWB
ZB = 8

PAD_N = 10240
ROWD = PAD_N // NS

RB = 1000
NG = N // RB

_mesh = plsc.VectorSubcoreMesh(core_axis_name="c", subcore_axis_name="s")


def _deg_body(dst3_hbm, out_hbm, zbuf_v, ones_v, didx_v, sem, isem, deg_sh):
    c = lax.axis_index("c")
    s = lax.axis_index("s")
    wid = c * NS + s

    def _fill(i, _):
        zbuf_v[pl.ds(i * 16, 16)] = jnp.zeros((16,), jnp.float32)
        ones_v[pl.ds(i * 16, 16)] = jnp.ones((16,), jnp.float32)
        return 0
    lax.fori_loop(0, ROWD // 16, _fill, 0)

    pltpu.sync_copy(zbuf_v, deg_sh.at[pl.ds(s * ROWD, ROWD)])

    plsc.subcore_barrier()

    pltpu.sync_copy(dst3_hbm.at[wid, 0], didx_v.at[0])
    pltpu.async_copy(dst3_hbm.at[wid, 1], didx_v.at[1], isem)

    def _edge(g, _):
        @pl.when(g + 2 < NCHUNK)
        def _():
            nxt = jnp.minimum(g + 2, NCHUNK - 1)
            pltpu.async_copy(dst3_hbm.at[wid, nxt],
                             didx_v.at[jnp.bitwise_and(nxt, 7)], isem)

        @pl.when(g >= 6)
        def _():
            pltpu.make_async_copy(ones_v.at[pl.ds(0, K)],
                                  deg_sh.at[didx_v.at[0]],
                                  sem.at[lax.rem(g, 6)]).wait()

        pltpu.async_copy(ones_v.at[pl.ds(0, K)],
                         deg_sh.at[didx_v.at[jnp.bitwise_and(g, 7)]],
                         sem.at[lax.rem(g, 6)], add=True)

        @pl.when(g + 1 < NCHUNK)
        def _():
            pltpu.make_async_copy(dst3_hbm.at[wid, 0], didx_v.at[0], isem).wait()
        return 0
    lax.fori_loop(0, NCHUNK, _edge, 0)

    def _drain(g, _):
        pltpu.make_async_copy(ones_v.at[pl.ds(0, K)], deg_sh.at[didx_v.at[0]],
                              sem.at[lax.rem(g, 6)]).wait()
        return 0
    lax.fori_loop(NCHUNK - 6, NCHUNK, _drain, 0)

    plsc.subcore_barrier()

    pltpu.sync_copy(deg_sh.at[pl.ds(s * ROWD, ROWD)],
                    out_hbm.at[c, pl.ds(s * ROWD, ROWD)])


_deg = pl.kernel(
    _deg_body,
    out_type=jax.ShapeDtypeStruct((NC, PAD_N), jnp.float32),
    mesh=_mesh,
    scratch_types=[
        pltpu.VMEM((ROWD,), jnp.float32),
        pltpu.VMEM((ROWD,), jnp.float32),
        pltpu.VMEM((8, K), jnp.int32),
        pltpu.SemaphoreType.DMA((6,)),
        pltpu.SemaphoreType.DMA,
        pltpu.VMEM_SHARED((PAD_N,), jnp.float32),
    ],
)


def _edge_body(y_hbm, src3_hbm, dst3_hbm, out_hbm,
               sidx_v, didx_v, rows_v, gsem, isem, ssem, acc_sh):
    c = lax.axis_index("c")
    s = lax.axis_index("s")
    wid = c * NS + s

    def _fill(i, _):
        rows_v[0, i // 8, pl.ds((i % 8) * 16, 16)] = jnp.zeros((16,), jnp.float32)
        return 0
    lax.fori_loop(0, K * (D // 16), _fill, 0)

    nfull = jnp.where(s == NS - 1, ROWL // K, ROWB // K)

    def _zero(i, _):
        pltpu.async_copy(rows_v.at[0],
                         acc_sh.at[pl.ds(s * ROWB + i * K, K)], isem)
        return 0
    lax.fori_loop(0, nfull, _zero, 0)

    @pl.when(s < NS - 1)
    def _():
        pltpu.async_copy(rows_v.at[0, pl.ds(0, ROWB - (ROWB // K) * K)],
                         acc_sh.at[pl.ds(s * ROWB + (ROWB // K) * K,
                                         ROWB - (ROWB // K) * K)], isem)

    @pl.when(s == NS - 1)
    def _():
        pltpu.async_copy(rows_v.at[0, pl.ds(0, ROWL - (ROWL // K) * K)],
                         acc_sh.at[pl.ds(s * ROWB + (ROWL // K) * K,
                                         ROWL - (ROWL // K) * K)], isem)

    def _zdrain(i, _):
        pltpu.make_async_copy(rows_v.at[0], acc_sh.at[pl.ds(0, K)], isem).wait()
        return 0
    lax.fori_loop(0, nfull, _zdrain, 0)

    @pl.when(s < NS - 1)
    def _():
        pltpu.make_async_copy(rows_v.at[0, pl.ds(0, ROWB - (ROWB // K) * K)],
                              acc_sh.at[pl.ds(0, ROWB - (ROWB // K) * K)],
                              isem).wait()

    @pl.when(s == NS - 1)
    def _():
        pltpu.make_async_copy(rows_v.at[0, pl.ds(0, ROWL - (ROWL // K) * K)],
                              acc_sh.at[pl.ds(0, ROWL - (ROWL // K) * K)],
                              isem).wait()

    plsc.subcore_barrier()

    def _idx_start(chunk):
        slot = jnp.bitwise_and(chunk, 7)
        pltpu.async_copy(src3_hbm.at[wid, chunk], sidx_v.at[slot], isem)
        pltpu.async_copy(dst3_hbm.at[wid, chunk], didx_v.at[slot], isem)

    def _idx_wait():
        pltpu.make_async_copy(src3_hbm.at[wid, 0], sidx_v.at[0], isem).wait()
        pltpu.make_async_copy(dst3_hbm.at[wid, 0], didx_v.at[0], isem).wait()

    def _gather_start(chunk):
        slot = jnp.bitwise_and(chunk, 3)
        pltpu.async_copy(y_hbm.at[sidx_v.at[jnp.bitwise_and(chunk, 7)]],
                         rows_v.at[slot], gsem.at[slot])

    def _gather_wait(chunk):
        pltpu.make_async_copy(y_hbm.at[sidx_v.at[0]], rows_v.at[0],
                              gsem.at[jnp.bitwise_and(chunk, 3)]).wait()

    def _scatter_start(chunk):
        slot = jnp.bitwise_and(chunk, 3)
        pltpu.async_copy(rows_v.at[slot],
                         acc_sh.at[didx_v.at[jnp.bitwise_and(chunk, 7)]],
                         ssem.at[slot], add=True)

    def _scatter_wait(chunk):
        pltpu.make_async_copy(y_hbm.at[sidx_v.at[0]], rows_v.at[0],
                              ssem.at[jnp.bitwise_and(chunk, 3)]).wait()

    _idx_start(0)
    _idx_wait()
    _gather_start(0)
    _idx_start(1)
    _idx_wait()
    _gather_start(1)
    _idx_start(2)

    def _edge(g, _):
        @pl.when(g + 2 < NCHUNK)
        def _():
            _idx_wait()

            @pl.when(g >= 2)
            def _():
                _scatter_wait(g - 2)
            _gather_start(jnp.minimum(g + 2, NCHUNK - 1))

        @pl.when(g + 3 < NCHUNK)
        def _():
            _idx_start(jnp.minimum(g + 3, NCHUNK - 1))

        _gather_wait(g)
        _scatter_start(g)
        return 0
    lax.fori_loop(0, NCHUNK, _edge, 0)

    _scatter_wait(NCHUNK - 4)
    _scatter_wait(NCHUNK - 3)
    _scatter_wait(NCHUNK - 2)
    _scatter_wait(NCHUNK - 1)

    plsc.subcore_barrier()

    @pl.when(s < NS - 1)
    def _():
        pltpu.sync_copy(acc_sh.at[pl.ds(s * ROWB, ROWB)],
                        out_hbm.at[c, pl.ds(s * ROWB, ROWB)])

    @pl.when(s == NS - 1)
    def _():
        pltpu.sync_copy(acc_sh.at[pl.ds(s * ROWB, ROWL)],
                        out_hbm.at[c, pl.ds(s * ROWB, ROWL)])


_edge_scatter = pl.kernel(
    _edge_body,
    out_type=jax.ShapeDtypeStruct((NC, N, D), jnp.float32),
    mesh=_mesh,
    scratch_types=[
        pltpu.VMEM((8, K), jnp.int32),
        pltpu.VMEM((8, K), jnp.int32),
        pltpu.VMEM((4, K, D), jnp.float32),
        pltpu.SemaphoreType.DMA((4,)),
        pltpu.SemaphoreType.DMA,
        pltpu.SemaphoreType.DMA((4,)),
        pltpu.VMEM_SHARED((N, D), jnp.float32),
    ],
)


def _dinv_of(degp_ref):
    deg = degp_ref[0, 0, 0, :] + degp_ref[1, 0, 0, :] + 1.0
    return lax.rsqrt(deg)


def _mmscale_body(x_ref, w_ref, degp_ref, y_ref):
    dinv = _dinv_of(degp_ref)
    y_ref[...] = jnp.dot(x_ref[...], w_ref[...],
                         preferred_element_type=jnp.float32) * dinv[:, None]


def _mmscale(x, w, degp4):
    return pl.pallas_call(
        _mmscale_body,
        grid=(NG,),
        in_specs=[
            pl.BlockSpec((RB, D), lambda i: (i, 0)),
            pl.BlockSpec((D, D), lambda i: (0, 0)),
            pl.BlockSpec((NC, 1, 1, RB), lambda i: (0, i, 0, 0)),
        ],
        out_specs=pl.BlockSpec((RB, D), lambda i: (i, 0)),
        out_shape=jax.ShapeDtypeStruct((N, D), jnp.float32),
    )(x, w, degp4)


def _mid_body(acc_ref, y_ref, degp_ref, b_ref, w_ref, o_ref):
    dinv = _dinv_of(degp_ref)
    tot = acc_ref[0] + acc_ref[1] + y_ref[...]
    h = jnp.maximum(tot * dinv[:, None] + b_ref[...], 0.0)
    o_ref[...] = jnp.dot(h, w_ref[...],
                         preferred_element_type=jnp.float32) * dinv[:, None]


def _mid(acc, y, degp4, b, w):
    return pl.pallas_call(
        _mid_body,
        grid=(NG,),
        in_specs=[
            pl.BlockSpec((NC, RB, D), lambda i: (0, i, 0)),
            pl.BlockSpec((RB, D), lambda i: (i, 0)),
            pl.BlockSpec((NC, 1, 1, RB), lambda i: (0, i, 0, 0)),
            pl.BlockSpec((1, D), lambda i: (0, 0)),
            pl.BlockSpec((D, D), lambda i: (0, 0)),
        ],
        out_specs=pl.BlockSpec((RB, D), lambda i: (i, 0)),
        out_shape=jax.ShapeDtypeStruct((N, D), jnp.float32),
    )(acc, y, degp4, b, w)


def _final_body(acc_ref, y_ref, degp_ref, b_ref, o_ref):
    dinv = _dinv_of(degp_ref)
    tot = acc_ref[0] + acc_ref[1] + y_ref[...]
    o_ref[...] = tot * dinv[:, None] + b_ref[...]


def _final(acc, y, degp4, b):
    return pl.pallas_call(
        _final_body,
        grid=(NG,),
        in_specs=[
            pl.BlockSpec((NC, RB, D), lambda i: (0, i, 0)),
            pl.BlockSpec((RB, D), lambda i: (i, 0)),
            pl.BlockSpec((NC, 1, 1, RB), lambda i: (0, i, 0, 0)),
            pl.BlockSpec((1, D), lambda i: (0, 0)),
        ],
        out_specs=pl.BlockSpec((RB, D), lambda i: (i, 0)),
        out_shape=jax.ShapeDtypeStruct((N, D), jnp.float32),
    )(acc, y, degp4, b)


def kernel(x, edge_index, W1, b1, W2, b2):
    src3 = edge_index[0].astype(jnp.int32).reshape(NW, NCHUNK, K)
    dst3 = edge_index[1].astype(jnp.int32).reshape(NW, NCHUNK, K)
    degp = _deg(dst3)[:, :N]
    degp4 = degp.reshape(NC, NG, 1, RB)
    y1 = _mmscale(x, W1, degp4)
    acc1 = _edge_scatter(y1, src3, dst3)
    y2 = _mid(acc1, y1, degp4, b1.reshape(1, D), W2)
    acc2 = _edge_scatter(y2, src3, dst3)
    return _final(acc2, y2, degp4, b2.reshape(1, D))


# speedup vs baseline: 36.6189x; 1.0369x over previous
import functools

import jax
import jax.numpy as jnp
from jax import lax
from jax.experimental import pallas as pl
from jax.experimental.pallas import tpu as pltpu
from jax.experimental.pallas import tpu_sc as plsc

N = 10000
D = 128
E = 320000

NC = 2
NS = 16
NW = NC * NS
EPW = E // NW
K = 80
NCHUNK = EPW // K

ROWB = 632
ROWL = N - (NS - 1) * ROWB
ZB = 8

PAD_N = 10240
ROWD = PAD_N // NS

RB = 1000
NG = N // RB

_mesh = plsc.VectorSubcoreMesh(core_axis_name="c", subcore_axis_name="s")


def _deg_body(ei4_hbm, out_hbm, zbuf_v, ones_v, didx_v, sem, isem, deg_sh):
    c = lax.axis_index("c")
    s = lax.axis_index("s")
    wid = c * NS + s

    def _fill(i, _):
        zbuf_v[pl.ds(i * 16, 16)] = jnp.zeros((16,), jnp.float32)
        ones_v[pl.ds(i * 16, 16)] = jnp.ones((16,), jnp.float32)
        return 0
    lax.fori_loop(0, ROWD // 16, _fill, 0)

    pltpu.sync_copy(zbuf_v, deg_sh.at[pl.ds(s * ROWD, ROWD)])

    plsc.subcore_barrier()

    pltpu.sync_copy(ei4_hbm.at[1, wid, 0], didx_v.at[0])
    pltpu.async_copy(ei4_hbm.at[1, wid, 1], didx_v.at[1], isem)

    def _edge(g, _):
        @pl.when(g + 2 < NCHUNK)
        def _():
            nxt = jnp.minimum(g + 2, NCHUNK - 1)
            pltpu.async_copy(ei4_hbm.at[1, wid, nxt],
                             didx_v.at[jnp.bitwise_and(nxt, 15)], isem)

        @pl.when(g >= 12)
        def _():
            pltpu.make_async_copy(ones_v.at[pl.ds(0, K)],
                                  deg_sh.at[didx_v.at[0]],
                                  sem.at[lax.rem(g, 12)]).wait()

        pltpu.async_copy(ones_v.at[pl.ds(0, K)],
                         deg_sh.at[didx_v.at[jnp.bitwise_and(g, 15)]],
                         sem.at[lax.rem(g, 12)], add=True)

        @pl.when(g + 1 < NCHUNK)
        def _():
            pltpu.make_async_copy(ei4_hbm.at[1, wid, 0], didx_v.at[0], isem).wait()
        return 0
    lax.fori_loop(0, NCHUNK, _edge, 0)

    def _drain(g, _):
        pltpu.make_async_copy(ones_v.at[pl.ds(0, K)], deg_sh.at[didx_v.at[0]],
                              sem.at[lax.rem(g, 12)]).wait()
        return 0
    lax.fori_loop(NCHUNK - 12, NCHUNK, _drain, 0)

    plsc.subcore_barrier()

    pltpu.sync_copy(deg_sh.at[pl.ds(s * ROWD, ROWD)],
                    out_hbm.at[c, pl.ds(s * ROWD, ROWD)])


_deg = pl.kernel(
    _deg_body,
    out_type=jax.ShapeDtypeStruct((NC, PAD_N), jnp.float32),
    mesh=_mesh,
    scratch_types=[
        pltpu.VMEM((ROWD,), jnp.float32),
        pltpu.VMEM((ROWD,), jnp.float32),
        pltpu.VMEM((16, K), jnp.int32),
        pltpu.SemaphoreType.DMA((12,)),
        pltpu.SemaphoreType.DMA,
        pltpu.VMEM_SHARED((PAD_N,), jnp.float32),
    ],
)


def _edge_body(y_hbm, ei4_hbm, out_hbm,
               sidx_v, didx_v, rows_v, gsem, isem, ssem, acc_sh):
    c = lax.axis_index("c")
    s = lax.axis_index("s")
    wid = c * NS + s

    def _fill(i, _):
        rows_v[0, i // 8, pl.ds((i % 8) * 16, 16)] = jnp.zeros((16,), jnp.float32)
        return 0
    lax.fori_loop(0, K * (D // 16), _fill, 0)

    nfull = jnp.where(s == NS - 1, ROWL // K, ROWB // K)

    def _zero(i, _):
        pltpu.async_copy(rows_v.at[0],
                         acc_sh.at[pl.ds(s * ROWB + i * K, K)], isem)
        return 0
    lax.fori_loop(0, nfull, _zero, 0)

    @pl.when(s < NS - 1)
    def _():
        pltpu.async_copy(rows_v.at[0, pl.ds(0, ROWB - (ROWB // K) * K)],
                         acc_sh.at[pl.ds(s * ROWB + (ROWB // K) * K,
                                         ROWB - (ROWB // K) * K)], isem)

    @pl.when(s == NS - 1)
    def _():
        pltpu.async_copy(rows_v.at[0, pl.ds(0, ROWL - (ROWL // K) * K)],
                         acc_sh.at[pl.ds(s * ROWB + (ROWL // K) * K,
                                         ROWL - (ROWL // K) * K)], isem)

    def _zdrain(i, _):
        pltpu.make_async_copy(rows_v.at[0], acc_sh.at[pl.ds(0, K)], isem).wait()
        return 0
    lax.fori_loop(0, nfull, _zdrain, 0)

    @pl.when(s < NS - 1)
    def _():
        pltpu.make_async_copy(rows_v.at[0, pl.ds(0, ROWB - (ROWB // K) * K)],
                              acc_sh.at[pl.ds(0, ROWB - (ROWB // K) * K)],
                              isem).wait()

    @pl.when(s == NS - 1)
    def _():
        pltpu.make_async_copy(rows_v.at[0, pl.ds(0, ROWL - (ROWL // K) * K)],
                              acc_sh.at[pl.ds(0, ROWL - (ROWL // K) * K)],
                              isem).wait()

    plsc.subcore_barrier()

    def _idx_start(chunk):
        slot = jnp.bitwise_and(chunk, 7)
        pltpu.async_copy(ei4_hbm.at[0, wid, chunk], sidx_v.at[slot], isem)
        pltpu.async_copy(ei4_hbm.at[1, wid, chunk], didx_v.at[slot], isem)

    def _idx_wait():
        pltpu.make_async_copy(ei4_hbm.at[0, wid, 0], sidx_v.at[0], isem).wait()
        pltpu.make_async_copy(ei4_hbm.at[1, wid, 0], didx_v.at[0], isem).wait()

    def _gather_start(chunk):
        slot = jnp.bitwise_and(chunk, 3)
        pltpu.async_copy(y_hbm.at[sidx_v.at[jnp.bitwise_and(chunk, 7)]],
                         rows_v.at[slot], gsem.at[slot])

    def _gather_wait(chunk):
        pltpu.make_async_copy(y_hbm.at[sidx_v.at[0]], rows_v.at[0],
                              gsem.at[jnp.bitwise_and(chunk, 3)]).wait()

    def _scatter_start(chunk):
        slot = jnp.bitwise_and(chunk, 3)
        pltpu.async_copy(rows_v.at[slot],
                         acc_sh.at[didx_v.at[jnp.bitwise_and(chunk, 7)]],
                         ssem.at[slot], add=True)

    def _scatter_wait(chunk):
        pltpu.make_async_copy(y_hbm.at[sidx_v.at[0]], rows_v.at[0],
                              ssem.at[jnp.bitwise_and(chunk, 3)]).wait()

    _idx_start(0)
    _idx_wait()
    _gather_start(0)
    _idx_start(1)
    _idx_wait()
    _gather_start(1)
    _idx_start(2)

    def _edge(g, _):
        @pl.when(g + 2 < NCHUNK)
        def _():
            _idx_wait()

            @pl.when(g >= 2)
            def _():
                _scatter_wait(g - 2)
            _gather_start(jnp.minimum(g + 2, NCHUNK - 1))

        @pl.when(g + 3 < NCHUNK)
        def _():
            _idx_start(jnp.minimum(g + 3, NCHUNK - 1))

        _gather_wait(g)
        _scatter_start(g)
        return 0
    lax.fori_loop(0, NCHUNK, _edge, 0)

    _scatter_wait(NCHUNK - 4)
    _scatter_wait(NCHUNK - 3)
    _scatter_wait(NCHUNK - 2)
    _scatter_wait(NCHUNK - 1)

    plsc.subcore_barrier()

    @pl.when(s < NS - 1)
    def _():
        pltpu.sync_copy(acc_sh.at[pl.ds(s * ROWB, ROWB)],
                        out_hbm.at[c, pl.ds(s * ROWB, ROWB)])

    @pl.when(s == NS - 1)
    def _():
        pltpu.sync_copy(acc_sh.at[pl.ds(s * ROWB, ROWL)],
                        out_hbm.at[c, pl.ds(s * ROWB, ROWL)])


_edge_scatter = pl.kernel(
    _edge_body,
    out_type=jax.ShapeDtypeStruct((NC, N, D), jnp.float32),
    mesh=_mesh,
    scratch_types=[
        pltpu.VMEM((8, K), jnp.int32),
        pltpu.VMEM((8, K), jnp.int32),
        pltpu.VMEM((4, K, D), jnp.float32),
        pltpu.SemaphoreType.DMA((4,)),
        pltpu.SemaphoreType.DMA,
        pltpu.SemaphoreType.DMA((4,)),
        pltpu.VMEM_SHARED((N, D), jnp.float32),
    ],
)


def _dinv_of(degp_ref):
    deg = degp_ref[0, 0, 0, :] + degp_ref[1, 0, 0, :] + 1.0
    return lax.rsqrt(deg)


def _mmscale_body(x_ref, w_ref, degp_ref, y_ref):
    dinv = _dinv_of(degp_ref)
    y_ref[...] = jnp.dot(x_ref[...], w_ref[...],
                         preferred_element_type=jnp.float32) * dinv[:, None]


def _mmscale(x, w, degp4):
    return pl.pallas_call(
        _mmscale_body,
        grid=(NG,),
        in_specs=[
            pl.BlockSpec((RB, D), lambda i: (i, 0)),
            pl.BlockSpec((D, D), lambda i: (0, 0)),
            pl.BlockSpec((NC, 1, 1, RB), lambda i: (0, i, 0, 0)),
        ],
        out_specs=pl.BlockSpec((RB, D), lambda i: (i, 0)),
        out_shape=jax.ShapeDtypeStruct((N, D), jnp.float32),
    )(x, w, degp4)


def _mid_body(acc_ref, y_ref, degp_ref, b_ref, w_ref, o_ref):
    dinv = _dinv_of(degp_ref)
    tot = acc_ref[0] + acc_ref[1] + y_ref[...]
    h = jnp.maximum(tot * dinv[:, None] + b_ref[...], 0.0)
    o_ref[...] = jnp.dot(h, w_ref[...],
                         preferred_element_type=jnp.float32) * dinv[:, None]


def _mid(acc, y, degp4, b, w):
    return pl.pallas_call(
        _mid_body,
        grid=(NG,),
        in_specs=[
            pl.BlockSpec((NC, RB, D), lambda i: (0, i, 0)),
            pl.BlockSpec((RB, D), lambda i: (i, 0)),
            pl.BlockSpec((NC, 1, 1, RB), lambda i: (0, i, 0, 0)),
            pl.BlockSpec((1, D), lambda i: (0, 0)),
            pl.BlockSpec((D, D), lambda i: (0, 0)),
        ],
        out_specs=pl.BlockSpec((RB, D), lambda i: (i, 0)),
        out_shape=jax.ShapeDtypeStruct((N, D), jnp.float32),
    )(acc, y, degp4, b, w)


def _final_body(acc_ref, y_ref, degp_ref, b_ref, o_ref):
    dinv = _dinv_of(degp_ref)
    tot = acc_ref[0] + acc_ref[1] + y_ref[...]
    o_ref[...] = tot * dinv[:, None] + b_ref[...]


def _final(acc, y, degp4, b):
    return pl.pallas_call(
        _final_body,
        grid=(NG,),
        in_specs=[
            pl.BlockSpec((NC, RB, D), lambda i: (0, i, 0)),
            pl.BlockSpec((RB, D), lambda i: (i, 0)),
            pl.BlockSpec((NC, 1, 1, RB), lambda i: (0, i, 0, 0)),
            pl.BlockSpec((1, D), lambda i: (0, 0)),
        ],
        out_specs=pl.BlockSpec((RB, D), lambda i: (i, 0)),
        out_shape=jax.ShapeDtypeStruct((N, D), jnp.float32),
    )(acc, y, degp4, b)


def kernel(x, edge_index, W1, b1, W2, b2):
    ei4 = edge_index.astype(jnp.int32).reshape(2, NW, NCHUNK, K)
    degp = _deg(ei4)[:, :N]
    degp4 = degp.reshape(NC, NG, 1, RB)
    y1 = _mmscale(x, W1, degp4)
    acc1 = _edge_scatter(y1, ei4)
    y2 = _mid(acc1, y1, degp4, b1.reshape(1, D), W2)
    acc2 = _edge_scatter(y2, ei4)
    return _final(acc2, y2, degp4, b2.reshape(1, D))


# speedup vs baseline: 37.5894x; 1.0265x over previous
import functools

import jax
import jax.numpy as jnp
from jax import lax
from jax.experimental import pallas as pl
from jax.experimental.pallas import tpu as pltpu
from jax.experimental.pallas import tpu_sc as plsc

N = 10000
D = 128
E = 320000

NC = 2
NS = 16
NW = NC * NS
EPW = E // NW
K = 80
NCHUNK = EPW // K

ROWB = 632
ROWL = N - (NS - 1) * ROWB
ZB = 8

PAD_N = 10240
ROWD = PAD_N // NS

RB = 1000
NG = N // RB

_mesh = plsc.VectorSubcoreMesh(core_axis_name="c", subcore_axis_name="s")


def _deg_body(ei_hbm, out_hbm, zbuf_v, ones_v, didx_v, sem, isem, deg_sh):
    c = lax.axis_index("c")
    s = lax.axis_index("s")
    wid = c * NS + s

    def _fill(i, _):
        zbuf_v[pl.ds(i * 16, 16)] = jnp.zeros((16,), jnp.float32)
        ones_v[pl.ds(i * 16, 16)] = jnp.ones((16,), jnp.float32)
        return 0
    lax.fori_loop(0, ROWD // 16, _fill, 0)

    pltpu.sync_copy(zbuf_v, deg_sh.at[pl.ds(s * ROWD, ROWD)])

    plsc.subcore_barrier()

    dbase = E + wid * EPW

    pltpu.sync_copy(ei_hbm.at[pl.ds(dbase, K)], didx_v.at[0])
    pltpu.async_copy(ei_hbm.at[pl.ds(dbase + K, K)], didx_v.at[1], isem)

    def _edge(g, _):
        @pl.when(g + 2 < NCHUNK)
        def _():
            nxt = jnp.minimum(g + 2, NCHUNK - 1)
            pltpu.async_copy(ei_hbm.at[pl.ds(dbase + nxt * K, K)],
                             didx_v.at[jnp.bitwise_and(nxt, 15)], isem)

        @pl.when(g >= 12)
        def _():
            pltpu.make_async_copy(ones_v.at[pl.ds(0, K)],
                                  deg_sh.at[didx_v.at[0]],
                                  sem.at[lax.rem(g, 12)]).wait()

        pltpu.async_copy(ones_v.at[pl.ds(0, K)],
                         deg_sh.at[didx_v.at[jnp.bitwise_and(g, 15)]],
                         sem.at[lax.rem(g, 12)], add=True)

        @pl.when(g + 1 < NCHUNK)
        def _():
            pltpu.make_async_copy(ei_hbm.at[pl.ds(dbase, K)], didx_v.at[0], isem).wait()
        return 0
    lax.fori_loop(0, NCHUNK, _edge, 0)

    def _drain(g, _):
        pltpu.make_async_copy(ones_v.at[pl.ds(0, K)], deg_sh.at[didx_v.at[0]],
                              sem.at[lax.rem(g, 12)]).wait()
        return 0
    lax.fori_loop(NCHUNK - 12, NCHUNK, _drain, 0)

    plsc.subcore_barrier()

    pltpu.sync_copy(deg_sh.at[pl.ds(s * ROWD, ROWD)],
                    out_hbm.at[c, pl.ds(s * ROWD, ROWD)])


_deg = pl.kernel(
    _deg_body,
    out_type=jax.ShapeDtypeStruct((NC, PAD_N), jnp.float32),
    mesh=_mesh,
    scratch_types=[
        pltpu.VMEM((ROWD,), jnp.float32),
        pltpu.VMEM((ROWD,), jnp.float32),
        pltpu.VMEM((16, K), jnp.int32),
        pltpu.SemaphoreType.DMA((12,)),
        pltpu.SemaphoreType.DMA,
        pltpu.VMEM_SHARED((PAD_N,), jnp.float32),
    ],
)


def _edge_body(y_hbm, ei_hbm, out_hbm,
               sidx_v, didx_v, rows_v, gsem, isem, ssem, acc_sh):
    c = lax.axis_index("c")
    s = lax.axis_index("s")
    wid = c * NS + s

    def _fill(i, _):
        rows_v[0, i // 8, pl.ds((i % 8) * 16, 16)] = jnp.zeros((16,), jnp.float32)
        return 0
    lax.fori_loop(0, K * (D // 16), _fill, 0)

    nfull = jnp.where(s == NS - 1, ROWL // K, ROWB // K)

    def _zero(i, _):
        pltpu.async_copy(rows_v.at[0],
                         acc_sh.at[pl.ds(s * ROWB + i * K, K)], isem)
        return 0
    lax.fori_loop(0, nfull, _zero, 0)

    @pl.when(s < NS - 1)
    def _():
        pltpu.async_copy(rows_v.at[0, pl.ds(0, ROWB - (ROWB // K) * K)],
                         acc_sh.at[pl.ds(s * ROWB + (ROWB // K) * K,
                                         ROWB - (ROWB // K) * K)], isem)

    @pl.when(s == NS - 1)
    def _():
        pltpu.async_copy(rows_v.at[0, pl.ds(0, ROWL - (ROWL // K) * K)],
                         acc_sh.at[pl.ds(s * ROWB + (ROWL // K) * K,
                                         ROWL - (ROWL // K) * K)], isem)

    def _zdrain(i, _):
        pltpu.make_async_copy(rows_v.at[0], acc_sh.at[pl.ds(0, K)], isem).wait()
        return 0
    lax.fori_loop(0, nfull, _zdrain, 0)

    @pl.when(s < NS - 1)
    def _():
        pltpu.make_async_copy(rows_v.at[0, pl.ds(0, ROWB - (ROWB // K) * K)],
                              acc_sh.at[pl.ds(0, ROWB - (ROWB // K) * K)],
                              isem).wait()

    @pl.when(s == NS - 1)
    def _():
        pltpu.make_async_copy(rows_v.at[0, pl.ds(0, ROWL - (ROWL // K) * K)],
                              acc_sh.at[pl.ds(0, ROWL - (ROWL // K) * K)],
                              isem).wait()

    plsc.subcore_barrier()

    def _idx_start(chunk):
        slot = jnp.bitwise_and(chunk, 7)
        pltpu.async_copy(ei_hbm.at[pl.ds(wid * EPW + chunk * K, K)],
                         sidx_v.at[slot], isem)
        pltpu.async_copy(ei_hbm.at[pl.ds(E + wid * EPW + chunk * K, K)],
                         didx_v.at[slot], isem)

    def _idx_wait():
        pltpu.make_async_copy(ei_hbm.at[pl.ds(0, K)], sidx_v.at[0], isem).wait()
        pltpu.make_async_copy(ei_hbm.at[pl.ds(0, K)], didx_v.at[0], isem).wait()

    def _gather_start(chunk):
        slot = jnp.bitwise_and(chunk, 3)
        pltpu.async_copy(y_hbm.at[sidx_v.at[jnp.bitwise_and(chunk, 7)]],
                         rows_v.at[slot], gsem.at[slot])

    def _gather_wait(chunk):
        pltpu.make_async_copy(y_hbm.at[sidx_v.at[0]], rows_v.at[0],
                              gsem.at[jnp.bitwise_and(chunk, 3)]).wait()

    def _scatter_start(chunk):
        slot = jnp.bitwise_and(chunk, 3)
        pltpu.async_copy(rows_v.at[slot],
                         acc_sh.at[didx_v.at[jnp.bitwise_and(chunk, 7)]],
                         ssem.at[slot], add=True)

    def _scatter_wait(chunk):
        pltpu.make_async_copy(y_hbm.at[sidx_v.at[0]], rows_v.at[0],
                              ssem.at[jnp.bitwise_and(chunk, 3)]).wait()

    _idx_start(0)
    _idx_wait()
    _gather_start(0)
    _idx_start(1)
    _idx_wait()
    _gather_start(1)
    _idx_start(2)

    def _edge(g, _):
        @pl.when(g + 2 < NCHUNK)
        def _():
            _idx_wait()

            @pl.when(g >= 2)
            def _():
                _scatter_wait(g - 2)
            _gather_start(jnp.minimum(g + 2, NCHUNK - 1))

        @pl.when(g + 3 < NCHUNK)
        def _():
            _idx_start(jnp.minimum(g + 3, NCHUNK - 1))

        _gather_wait(g)
        _scatter_start(g)
        return 0
    lax.fori_loop(0, NCHUNK, _edge, 0)

    _scatter_wait(NCHUNK - 4)
    _scatter_wait(NCHUNK - 3)
    _scatter_wait(NCHUNK - 2)
    _scatter_wait(NCHUNK - 1)

    plsc.subcore_barrier()

    @pl.when(s < NS - 1)
    def _():
        pltpu.sync_copy(acc_sh.at[pl.ds(s * ROWB, ROWB)],
                        out_hbm.at[c, pl.ds(s * ROWB, ROWB)])

    @pl.when(s == NS - 1)
    def _():
        pltpu.sync_copy(acc_sh.at[pl.ds(s * ROWB, ROWL)],
                        out_hbm.at[c, pl.ds(s * ROWB, ROWL)])


_edge_scatter = pl.kernel(
    _edge_body,
    out_type=jax.ShapeDtypeStruct((NC, N, D), jnp.float32),
    mesh=_mesh,
    scratch_types=[
        pltpu.VMEM((8, K), jnp.int32),
        pltpu.VMEM((8, K), jnp.int32),
        pltpu.VMEM((4, K, D), jnp.float32),
        pltpu.SemaphoreType.DMA((4,)),
        pltpu.SemaphoreType.DMA,
        pltpu.SemaphoreType.DMA((4,)),
        pltpu.VMEM_SHARED((N, D), jnp.float32),
    ],
)


def _dinv_of(degp_ref):
    deg = degp_ref[0, 0, 0, :] + degp_ref[1, 0, 0, :] + 1.0
    return lax.rsqrt(deg)


def _mmscale_body(x_ref, w_ref, degp_ref, y_ref):
    dinv = _dinv_of(degp_ref)
    y_ref[...] = jnp.dot(x_ref[...], w_ref[...],
                         preferred_element_type=jnp.float32) * dinv[:, None]


def _mmscale(x, w, degp4):
    return pl.pallas_call(
        _mmscale_body,
        grid=(NG,),
        in_specs=[
            pl.BlockSpec((RB, D), lambda i: (i, 0)),
            pl.BlockSpec((D, D), lambda i: (0, 0)),
            pl.BlockSpec((NC, 1, 1, RB), lambda i: (0, i, 0, 0)),
        ],
        out_specs=pl.BlockSpec((RB, D), lambda i: (i, 0)),
        out_shape=jax.ShapeDtypeStruct((N, D), jnp.float32),
    )(x, w, degp4)


def _mid_body(acc_ref, y_ref, degp_ref, b_ref, w_ref, o_ref):
    dinv = _dinv_of(degp_ref)
    tot = acc_ref[0] + acc_ref[1] + y_ref[...]
    h = jnp.maximum(tot * dinv[:, None] + b_ref[...], 0.0)
    o_ref[...] = jnp.dot(h, w_ref[...],
                         preferred_element_type=jnp.float32) * dinv[:, None]


def _mid(acc, y, degp4, b, w):
    return pl.pallas_call(
        _mid_body,
        grid=(NG,),
        in_specs=[
            pl.BlockSpec((NC, RB, D), lambda i: (0, i, 0)),
            pl.BlockSpec((RB, D), lambda i: (i, 0)),
            pl.BlockSpec((NC, 1, 1, RB), lambda i: (0, i, 0, 0)),
            pl.BlockSpec((1, D), lambda i: (0, 0)),
            pl.BlockSpec((D, D), lambda i: (0, 0)),
        ],
        out_specs=pl.BlockSpec((RB, D), lambda i: (i, 0)),
        out_shape=jax.ShapeDtypeStruct((N, D), jnp.float32),
    )(acc, y, degp4, b, w)


def _final_body(acc_ref, y_ref, degp_ref, b_ref, o_ref):
    dinv = _dinv_of(degp_ref)
    tot = acc_ref[0] + acc_ref[1] + y_ref[...]
    o_ref[...] = tot * dinv[:, None] + b_ref[...]


def _final(acc, y, degp4, b):
    return pl.pallas_call(
        _final_body,
        grid=(NG,),
        in_specs=[
            pl.BlockSpec((NC, RB, D), lambda i: (0, i, 0)),
            pl.BlockSpec((RB, D), lambda i: (i, 0)),
            pl.BlockSpec((NC, 1, 1, RB), lambda i: (0, i, 0, 0)),
            pl.BlockSpec((1, D), lambda i: (0, 0)),
        ],
        out_specs=pl.BlockSpec((RB, D), lambda i: (i, 0)),
        out_shape=jax.ShapeDtypeStruct((N, D), jnp.float32),
    )(acc, y, degp4, b)


def kernel(x, edge_index, W1, b1, W2, b2):
    ei = edge_index.astype(jnp.int32).reshape(2 * E)
    degp = _deg(ei)[:, :N]
    degp4 = degp.reshape(NC, NG, 1, RB)
    y1 = _mmscale(x, W1, degp4)
    acc1 = _edge_scatter(y1, ei)
    y2 = _mid(acc1, y1, degp4, b1.reshape(1, D), W2)
    acc2 = _edge_scatter(y2, ei)
    return _final(acc2, y2, degp4, b2.reshape(1, D))


# speedup vs baseline: 38.5143x; 1.0246x over previous
import functools

import jax
import jax.numpy as jnp
from jax import lax
from jax.experimental import pallas as pl
from jax.experimental.pallas import tpu as pltpu
from jax.experimental.pallas import tpu_sc as plsc

N = 10000
D = 128
E = 320000

NC = 2
NS = 16
NW = NC * NS
EPW = E // NW
K = 80
NCHUNK = EPW // K

ROWB = 632
ROWL = N - (NS - 1) * ROWB
ZB = 8

PAD_N = 10240
ROWD = PAD_N // NS

RB = 2000
NG = N // RB

_mesh = plsc.VectorSubcoreMesh(core_axis_name="c", subcore_axis_name="s")


def _deg_body(ei_hbm, out_hbm, zbuf_v, ones_v, didx_v, sem, isem, deg_sh):
    c = lax.axis_index("c")
    s = lax.axis_index("s")
    wid = c * NS + s

    def _fill(i, _):
        zbuf_v[pl.ds(i * 16, 16)] = jnp.zeros((16,), jnp.float32)
        ones_v[pl.ds(i * 16, 16)] = jnp.ones((16,), jnp.float32)
        return 0
    lax.fori_loop(0, ROWD // 16, _fill, 0)

    pltpu.sync_copy(zbuf_v, deg_sh.at[pl.ds(s * ROWD, ROWD)])

    plsc.subcore_barrier()

    dbase = E + wid * EPW

    pltpu.sync_copy(ei_hbm.at[pl.ds(dbase, K)], didx_v.at[0])
    pltpu.async_copy(ei_hbm.at[pl.ds(dbase + K, K)], didx_v.at[1], isem)

    def _edge(g, _):
        @pl.when(g + 2 < NCHUNK)
        def _():
            nxt = jnp.minimum(g + 2, NCHUNK - 1)
            pltpu.async_copy(ei_hbm.at[pl.ds(dbase + nxt * K, K)],
                             didx_v.at[jnp.bitwise_and(nxt, 15)], isem)

        @pl.when(g >= 12)
        def _():
            pltpu.make_async_copy(ones_v.at[pl.ds(0, K)],
                                  deg_sh.at[didx_v.at[0]],
                                  sem.at[lax.rem(g, 12)]).wait()

        pltpu.async_copy(ones_v.at[pl.ds(0, K)],
                         deg_sh.at[didx_v.at[jnp.bitwise_and(g, 15)]],
                         sem.at[lax.rem(g, 12)], add=True)

        @pl.when(g + 1 < NCHUNK)
        def _():
            pltpu.make_async_copy(ei_hbm.at[pl.ds(dbase, K)], didx_v.at[0], isem).wait()
        return 0
    lax.fori_loop(0, NCHUNK, _edge, 0)

    def _drain(g, _):
        pltpu.make_async_copy(ones_v.at[pl.ds(0, K)], deg_sh.at[didx_v.at[0]],
                              sem.at[lax.rem(g, 12)]).wait()
        return 0
    lax.fori_loop(NCHUNK - 12, NCHUNK, _drain, 0)

    plsc.subcore_barrier()

    pltpu.sync_copy(deg_sh.at[pl.ds(s * ROWD, ROWD)],
                    out_hbm.at[c, pl.ds(s * ROWD, ROWD)])


_deg = pl.kernel(
    _deg_body,
    out_type=jax.ShapeDtypeStruct((NC, PAD_N), jnp.float32),
    mesh=_mesh,
    scratch_types=[
        pltpu.VMEM((ROWD,), jnp.float32),
        pltpu.VMEM((ROWD,), jnp.float32),
        pltpu.VMEM((16, K), jnp.int32),
        pltpu.SemaphoreType.DMA((12,)),
        pltpu.SemaphoreType.DMA,
        pltpu.VMEM_SHARED((PAD_N,), jnp.float32),
    ],
)


def _edge_body(y_hbm, ei_hbm, out_hbm,
               sidx_v, didx_v, rows_v, gsem, isem, ssem, acc_sh):
    c = lax.axis_index("c")
    s = lax.axis_index("s")
    wid = c * NS + s

    def _fill(i, _):
        rows_v[0, i // 8, pl.ds((i % 8) * 16, 16)] = jnp.zeros((16,), jnp.float32)
        return 0
    lax.fori_loop(0, K * (D // 16), _fill, 0)

    nfull = jnp.where(s == NS - 1, ROWL // K, ROWB // K)

    def _zero(i, _):
        pltpu.async_copy(rows_v.at[0],
                         acc_sh.at[pl.ds(s * ROWB + i * K, K)], isem)
        return 0
    lax.fori_loop(0, nfull, _zero, 0)

    @pl.when(s < NS - 1)
    def _():
        pltpu.async_copy(rows_v.at[0, pl.ds(0, ROWB - (ROWB // K) * K)],
                         acc_sh.at[pl.ds(s * ROWB + (ROWB // K) * K,
                                         ROWB - (ROWB // K) * K)], isem)

    @pl.when(s == NS - 1)
    def _():
        pltpu.async_copy(rows_v.at[0, pl.ds(0, ROWL - (ROWL // K) * K)],
                         acc_sh.at[pl.ds(s * ROWB + (ROWL // K) * K,
                                         ROWL - (ROWL // K) * K)], isem)

    def _zdrain(i, _):
        pltpu.make_async_copy(rows_v.at[0], acc_sh.at[pl.ds(0, K)], isem).wait()
        return 0
    lax.fori_loop(0, nfull, _zdrain, 0)

    @pl.when(s < NS - 1)
    def _():
        pltpu.make_async_copy(rows_v.at[0, pl.ds(0, ROWB - (ROWB // K) * K)],
                              acc_sh.at[pl.ds(0, ROWB - (ROWB // K) * K)],
                              isem).wait()

    @pl.when(s == NS - 1)
    def _():
        pltpu.make_async_copy(rows_v.at[0, pl.ds(0, ROWL - (ROWL // K) * K)],
                              acc_sh.at[pl.ds(0, ROWL - (ROWL // K) * K)],
                              isem).wait()

    plsc.subcore_barrier()

    def _idx_start(chunk):
        slot = jnp.bitwise_and(chunk, 7)
        pltpu.async_copy(ei_hbm.at[pl.ds(wid * EPW + chunk * K, K)],
                         sidx_v.at[slot], isem)
        pltpu.async_copy(ei_hbm.at[pl.ds(E + wid * EPW + chunk * K, K)],
                         didx_v.at[slot], isem)

    def _idx_wait():
        pltpu.make_async_copy(ei_hbm.at[pl.ds(0, K)], sidx_v.at[0], isem).wait()
        pltpu.make_async_copy(ei_hbm.at[pl.ds(0, K)], didx_v.at[0], isem).wait()

    def _gather_start(chunk):
        slot = jnp.bitwise_and(chunk, 3)
        pltpu.async_copy(y_hbm.at[sidx_v.at[jnp.bitwise_and(chunk, 7)]],
                         rows_v.at[slot], gsem.at[slot])

    def _gather_wait(chunk):
        pltpu.make_async_copy(y_hbm.at[sidx_v.at[0]], rows_v.at[0],
                              gsem.at[jnp.bitwise_and(chunk, 3)]).wait()

    def _scatter_start(chunk):
        slot = jnp.bitwise_and(chunk, 3)
        pltpu.async_copy(rows_v.at[slot],
                         acc_sh.at[didx_v.at[jnp.bitwise_and(chunk, 7)]],
                         ssem.at[slot], add=True)

    def _scatter_wait(chunk):
        pltpu.make_async_copy(y_hbm.at[sidx_v.at[0]], rows_v.at[0],
                              ssem.at[jnp.bitwise_and(chunk, 3)]).wait()

    _idx_start(0)
    _idx_wait()
    _gather_start(0)
    _idx_start(1)
    _idx_wait()
    _gather_start(1)
    _idx_start(2)

    def _edge(g, _):
        @pl.when(g + 2 < NCHUNK)
        def _():
            _idx_wait()

            @pl.when(g >= 2)
            def _():
                _scatter_wait(g - 2)
            _gather_start(jnp.minimum(g + 2, NCHUNK - 1))

        @pl.when(g + 3 < NCHUNK)
        def _():
            _idx_start(jnp.minimum(g + 3, NCHUNK - 1))

        _gather_wait(g)
        _scatter_start(g)
        return 0
    lax.fori_loop(0, NCHUNK, _edge, 0)

    _scatter_wait(NCHUNK - 4)
    _scatter_wait(NCHUNK - 3)
    _scatter_wait(NCHUNK - 2)
    _scatter_wait(NCHUNK - 1)

    plsc.subcore_barrier()

    @pl.when(s < NS - 1)
    def _():
        pltpu.sync_copy(acc_sh.at[pl.ds(s * ROWB, ROWB)],
                        out_hbm.at[c, pl.ds(s * ROWB, ROWB)])

    @pl.when(s == NS - 1)
    def _():
        pltpu.sync_copy(acc_sh.at[pl.ds(s * ROWB, ROWL)],
                        out_hbm.at[c, pl.ds(s * ROWB, ROWL)])


_edge_scatter = pl.kernel(
    _edge_body,
    out_type=jax.ShapeDtypeStruct((NC, N, D), jnp.float32),
    mesh=_mesh,
    scratch_types=[
        pltpu.VMEM((8, K), jnp.int32),
        pltpu.VMEM((8, K), jnp.int32),
        pltpu.VMEM((4, K, D), jnp.float32),
        pltpu.SemaphoreType.DMA((4,)),
        pltpu.SemaphoreType.DMA,
        pltpu.SemaphoreType.DMA((4,)),
        pltpu.VMEM_SHARED((N, D), jnp.float32),
    ],
)


def _dinv_of(degp_ref):
    deg = degp_ref[0, 0, 0, :] + degp_ref[1, 0, 0, :] + 1.0
    return lax.rsqrt(deg)


def _mmscale_body(x_ref, w_ref, degp_ref, y_ref):
    dinv = _dinv_of(degp_ref)
    y_ref[...] = jnp.dot(x_ref[...], w_ref[...],
                         preferred_element_type=jnp.float32) * dinv[:, None]


def _mmscale(x, w, degp4):
    return pl.pallas_call(
        _mmscale_body,
        grid=(NG,),
        in_specs=[
            pl.BlockSpec((RB, D), lambda i: (i, 0)),
            pl.BlockSpec((D, D), lambda i: (0, 0)),
            pl.BlockSpec((NC, 1, 1, RB), lambda i: (0, i, 0, 0)),
        ],
        out_specs=pl.BlockSpec((RB, D), lambda i: (i, 0)),
        out_shape=jax.ShapeDtypeStruct((N, D), jnp.float32),
    )(x, w, degp4)


def _mid_body(acc_ref, y_ref, degp_ref, b_ref, w_ref, o_ref):
    dinv = _dinv_of(degp_ref)
    tot = acc_ref[0] + acc_ref[1] + y_ref[...]
    h = jnp.maximum(tot * dinv[:, None] + b_ref[...], 0.0)
    o_ref[...] = jnp.dot(h, w_ref[...],
                         preferred_element_type=jnp.float32) * dinv[:, None]


def _mid(acc, y, degp4, b, w):
    return pl.pallas_call(
        _mid_body,
        grid=(NG,),
        in_specs=[
            pl.BlockSpec((NC, RB, D), lambda i: (0, i, 0)),
            pl.BlockSpec((RB, D), lambda i: (i, 0)),
            pl.BlockSpec((NC, 1, 1, RB), lambda i: (0, i, 0, 0)),
            pl.BlockSpec((1, D), lambda i: (0, 0)),
            pl.BlockSpec((D, D), lambda i: (0, 0)),
        ],
        out_specs=pl.BlockSpec((RB, D), lambda i: (i, 0)),
        out_shape=jax.ShapeDtypeStruct((N, D), jnp.float32),
    )(acc, y, degp4, b, w)


def _final_body(acc_ref, y_ref, degp_ref, b_ref, o_ref):
    dinv = _dinv_of(degp_ref)
    tot = acc_ref[0] + acc_ref[1] + y_ref[...]
    o_ref[...] = tot * dinv[:, None] + b_ref[...]


def _final(acc, y, degp4, b):
    return pl.pallas_call(
        _final_body,
        grid=(NG,),
        in_specs=[
            pl.BlockSpec((NC, RB, D), lambda i: (0, i, 0)),
            pl.BlockSpec((RB, D), lambda i: (i, 0)),
            pl.BlockSpec((NC, 1, 1, RB), lambda i: (0, i, 0, 0)),
            pl.BlockSpec((1, D), lambda i: (0, 0)),
        ],
        out_specs=pl.BlockSpec((RB, D), lambda i: (i, 0)),
        out_shape=jax.ShapeDtypeStruct((N, D), jnp.float32),
    )(acc, y, degp4, b)


def kernel(x, edge_index, W1, b1, W2, b2):
    ei = edge_index.astype(jnp.int32).reshape(2 * E)
    degp = _deg(ei)[:, :N]
    degp4 = degp.reshape(NC, NG, 1, RB)
    y1 = _mmscale(x, W1, degp4)
    acc1 = _edge_scatter(y1, ei)
    y2 = _mid(acc1, y1, degp4, b1.reshape(1, D), W2)
    acc2 = _edge_scatter(y2, ei)
    return _final(acc2, y2, degp4, b2.reshape(1, D))


# speedup vs baseline: 39.2501x; 1.0191x over previous
import functools

import jax
import jax.numpy as jnp
from jax import lax
from jax.experimental import pallas as pl
from jax.experimental.pallas import tpu as pltpu
from jax.experimental.pallas import tpu_sc as plsc

N = 10000
D = 128
E = 320000

NC = 2
NS = 16
NW = NC * NS
EPW = E // NW
K = 80
NCHUNK = EPW // K
KD = 128
NCHD = EPW // KD

ROWB = 632
ROWL = N - (NS - 1) * ROWB
ZB = 8

PAD_N = 10240
ROWD = PAD_N // NS

RB = 2000
NG = N // RB

_mesh = plsc.VectorSubcoreMesh(core_axis_name="c", subcore_axis_name="s")


def _deg_body(ei_hbm, out_hbm, zbuf_v, ones_v, didx_v, tidx_v, sem, isem, deg_sh):
    c = lax.axis_index("c")
    s = lax.axis_index("s")
    wid = c * NS + s

    def _fill(i, _):
        zbuf_v[pl.ds(i * 16, 16)] = jnp.zeros((16,), jnp.float32)
        ones_v[pl.ds(i * 16, 16)] = jnp.ones((16,), jnp.float32)
        return 0
    lax.fori_loop(0, ROWD // 16, _fill, 0)

    pltpu.sync_copy(zbuf_v, deg_sh.at[pl.ds(s * ROWD, ROWD)])

    plsc.subcore_barrier()

    dbase = E + wid * EPW

    pltpu.sync_copy(ei_hbm.at[pl.ds(dbase, KD)], didx_v.at[0])
    pltpu.async_copy(ei_hbm.at[pl.ds(dbase + KD, KD)], didx_v.at[1], isem)

    def _edge(g, _):
        @pl.when(g + 2 < NCHD)
        def _():
            nxt = jnp.minimum(g + 2, NCHD - 1)
            pltpu.async_copy(ei_hbm.at[pl.ds(dbase + nxt * KD, KD)],
                             didx_v.at[jnp.bitwise_and(nxt, 15)], isem)

        @pl.when(g >= 12)
        def _():
            pltpu.make_async_copy(ones_v.at[pl.ds(0, KD)],
                                  deg_sh.at[didx_v.at[0]],
                                  sem.at[lax.rem(g, 12)]).wait()

        pltpu.async_copy(ones_v.at[pl.ds(0, KD)],
                         deg_sh.at[didx_v.at[jnp.bitwise_and(g, 15)]],
                         sem.at[lax.rem(g, 12)], add=True)

        @pl.when(g + 1 < NCHD)
        def _():
            pltpu.make_async_copy(ei_hbm.at[pl.ds(dbase, KD)], didx_v.at[0],
                                  isem).wait()
        return 0
    lax.fori_loop(0, NCHD, _edge, 0)

    def _drain(g, _):
        pltpu.make_async_copy(ones_v.at[pl.ds(0, KD)], deg_sh.at[didx_v.at[0]],
                              sem.at[lax.rem(g, 12)]).wait()
        return 0
    lax.fori_loop(NCHD - 12, NCHD, _drain, 0)

    pltpu.sync_copy(ei_hbm.at[pl.ds(dbase + NCHD * KD, EPW - NCHD * KD)], tidx_v)
    pltpu.sync_copy(ones_v.at[pl.ds(0, EPW - NCHD * KD)],
                    deg_sh.at[tidx_v], add=True)

    plsc.subcore_barrier()

    pltpu.sync_copy(deg_sh.at[pl.ds(s * ROWD, ROWD)],
                    out_hbm.at[c, pl.ds(s * ROWD, ROWD)])


_deg = pl.kernel(
    _deg_body,
    out_type=jax.ShapeDtypeStruct((NC, PAD_N), jnp.float32),
    mesh=_mesh,
    scratch_types=[
        pltpu.VMEM((ROWD,), jnp.float32),
        pltpu.VMEM((ROWD,), jnp.float32),
        pltpu.VMEM((16, KD), jnp.int32),
        pltpu.VMEM((EPW - (EPW // KD) * KD,), jnp.int32),
        pltpu.SemaphoreType.DMA((12,)),
        pltpu.SemaphoreType.DMA,
        pltpu.VMEM_SHARED((PAD_N,), jnp.float32),
    ],
)


def _edge_body(y_hbm, ei_hbm, out_hbm,
               sidx_v, didx_v, rows_v, gsem, isem, ssem, acc_sh):
    c = lax.axis_index("c")
    s = lax.axis_index("s")
    wid = c * NS + s

    def _fill(i, _):
        rows_v[0, i // 8, pl.ds((i % 8) * 16, 16)] = jnp.zeros((16,), jnp.float32)
        return 0
    lax.fori_loop(0, K * (D // 16), _fill, 0)

    nfull = jnp.where(s == NS - 1, ROWL // K, ROWB // K)

    def _zero(i, _):
        pltpu.async_copy(rows_v.at[0],
                         acc_sh.at[pl.ds(s * ROWB + i * K, K)], isem)
        return 0
    lax.fori_loop(0, nfull, _zero, 0)

    @pl.when(s < NS - 1)
    def _():
        pltpu.async_copy(rows_v.at[0, pl.ds(0, ROWB - (ROWB // K) * K)],
                         acc_sh.at[pl.ds(s * ROWB + (ROWB // K) * K,
                                         ROWB - (ROWB // K) * K)], isem)

    @pl.when(s == NS - 1)
    def _():
        pltpu.async_copy(rows_v.at[0, pl.ds(0, ROWL - (ROWL // K) * K)],
                         acc_sh.at[pl.ds(s * ROWB + (ROWL // K) * K,
                                         ROWL - (ROWL // K) * K)], isem)

    def _zdrain(i, _):
        pltpu.make_async_copy(rows_v.at[0], acc_sh.at[pl.ds(0, K)], isem).wait()
        return 0
    lax.fori_loop(0, nfull, _zdrain, 0)

    @pl.when(s < NS - 1)
    def _():
        pltpu.make_async_copy(rows_v.at[0, pl.ds(0, ROWB - (ROWB // K) * K)],
                              acc_sh.at[pl.ds(0, ROWB - (ROWB // K) * K)],
                              isem).wait()

    @pl.when(s == NS - 1)
    def _():
        pltpu.make_async_copy(rows_v.at[0, pl.ds(0, ROWL - (ROWL // K) * K)],
                              acc_sh.at[pl.ds(0, ROWL - (ROWL // K) * K)],
                              isem).wait()

    plsc.subcore_barrier()

    def _idx_start(chunk):
        slot = jnp.bitwise_and(chunk, 7)
        pltpu.async_copy(ei_hbm.at[pl.ds(wid * EPW + chunk * K, K)],
                         sidx_v.at[slot], isem)
        pltpu.async_copy(ei_hbm.at[pl.ds(E + wid * EPW + chunk * K, K)],
                         didx_v.at[slot], isem)

    def _idx_wait():
        pltpu.make_async_copy(ei_hbm.at[pl.ds(0, K)], sidx_v.at[0], isem).wait()
        pltpu.make_async_copy(ei_hbm.at[pl.ds(0, K)], didx_v.at[0], isem).wait()

    def _gather_start(chunk):
        slot = jnp.bitwise_and(chunk, 3)
        pltpu.async_copy(y_hbm.at[sidx_v.at[jnp.bitwise_and(chunk, 7)]],
                         rows_v.at[slot], gsem.at[slot])

    def _gather_wait(chunk):
        pltpu.make_async_copy(y_hbm.at[sidx_v.at[0]], rows_v.at[0],
                              gsem.at[jnp.bitwise_and(chunk, 3)]).wait()

    def _scatter_start(chunk):
        slot = jnp.bitwise_and(chunk, 3)
        pltpu.async_copy(rows_v.at[slot],
                         acc_sh.at[didx_v.at[jnp.bitwise_and(chunk, 7)]],
                         ssem.at[slot], add=True)

    def _scatter_wait(chunk):
        pltpu.make_async_copy(y_hbm.at[sidx_v.at[0]], rows_v.at[0],
                              ssem.at[jnp.bitwise_and(chunk, 3)]).wait()

    _idx_start(0)
    _idx_wait()
    _gather_start(0)
    _idx_start(1)
    _idx_wait()
    _gather_start(1)
    _idx_start(2)

    def _edge(g, _):
        @pl.when(g + 2 < NCHUNK)
        def _():
            _idx_wait()

            @pl.when(g >= 2)
            def _():
                _scatter_wait(g - 2)
            _gather_start(jnp.minimum(g + 2, NCHUNK - 1))

        @pl.when(g + 3 < NCHUNK)
        def _():
            _idx_start(jnp.minimum(g + 3, NCHUNK - 1))

        _gather_wait(g)
        _scatter_start(g)
        return 0
    lax.fori_loop(0, NCHUNK, _edge, 0)

    _scatter_wait(NCHUNK - 4)
    _scatter_wait(NCHUNK - 3)
    _scatter_wait(NCHUNK - 2)
    _scatter_wait(NCHUNK - 1)

    plsc.subcore_barrier()

    @pl.when(s < NS - 1)
    def _():
        pltpu.sync_copy(acc_sh.at[pl.ds(s * ROWB, ROWB)],
                        out_hbm.at[c, pl.ds(s * ROWB, ROWB)])

    @pl.when(s == NS - 1)
    def _():
        pltpu.sync_copy(acc_sh.at[pl.ds(s * ROWB, ROWL)],
                        out_hbm.at[c, pl.ds(s * ROWB, ROWL)])


_edge_scatter = pl.kernel(
    _edge_body,
    out_type=jax.ShapeDtypeStruct((NC, N, D), jnp.float32),
    mesh=_mesh,
    scratch_types=[
        pltpu.VMEM((8, K), jnp.int32),
        pltpu.VMEM((8, K), jnp.int32),
        pltpu.VMEM((4, K, D), jnp.float32),
        pltpu.SemaphoreType.DMA((4,)),
        pltpu.SemaphoreType.DMA,
        pltpu.SemaphoreType.DMA((4,)),
        pltpu.VMEM_SHARED((N, D), jnp.float32),
    ],
)


def _dinv_of(degp_ref):
    deg = degp_ref[0, 0, 0, :] + degp_ref[1, 0, 0, :] + 1.0
    return lax.rsqrt(deg)


def _mmscale_body(x_ref, w_ref, degp_ref, y_ref):
    dinv = _dinv_of(degp_ref)
    y_ref[...] = jnp.dot(x_ref[...], w_ref[...],
                         preferred_element_type=jnp.float32) * dinv[:, None]


def _mmscale(x, w, degp4):
    return pl.pallas_call(
        _mmscale_body,
        grid=(NG,),
        in_specs=[
            pl.BlockSpec((RB, D), lambda i: (i, 0)),
            pl.BlockSpec((D, D), lambda i: (0, 0)),
            pl.BlockSpec((NC, 1, 1, RB), lambda i: (0, i, 0, 0)),
        ],
        out_specs=pl.BlockSpec((RB, D), lambda i: (i, 0)),
        out_shape=jax.ShapeDtypeStruct((N, D), jnp.float32),
    )(x, w, degp4)


def _mid_body(acc_ref, y_ref, degp_ref, b_ref, w_ref, o_ref):
    dinv = _dinv_of(degp_ref)
    tot = acc_ref[0] + acc_ref[1] + y_ref[...]
    h = jnp.maximum(tot * dinv[:, None] + b_ref[...], 0.0)
    o_ref[...] = jnp.dot(h, w_ref[...],
                         preferred_element_type=jnp.float32) * dinv[:, None]


def _mid(acc, y, degp4, b, w):
    return pl.pallas_call(
        _mid_body,
        grid=(NG,),
        in_specs=[
            pl.BlockSpec((NC, RB, D), lambda i: (0, i, 0)),
            pl.BlockSpec((RB, D), lambda i: (i, 0)),
            pl.BlockSpec((NC, 1, 1, RB), lambda i: (0, i, 0, 0)),
            pl.BlockSpec((1, D), lambda i: (0, 0)),
            pl.BlockSpec((D, D), lambda i: (0, 0)),
        ],
        out_specs=pl.BlockSpec((RB, D), lambda i: (i, 0)),
        out_shape=jax.ShapeDtypeStruct((N, D), jnp.float32),
    )(acc, y, degp4, b, w)


def _final_body(acc_ref, y_ref, degp_ref, b_ref, o_ref):
    dinv = _dinv_of(degp_ref)
    tot = acc_ref[0] + acc_ref[1] + y_ref[...]
    o_ref[...] = tot * dinv[:, None] + b_ref[...]


def _final(acc, y, degp4, b):
    return pl.pallas_call(
        _final_body,
        grid=(NG,),
        in_specs=[
            pl.BlockSpec((NC, RB, D), lambda i: (0, i, 0)),
            pl.BlockSpec((RB, D), lambda i: (i, 0)),
            pl.BlockSpec((NC, 1, 1, RB), lambda i: (0, i, 0, 0)),
            pl.BlockSpec((1, D), lambda i: (0, 0)),
        ],
        out_specs=pl.BlockSpec((RB, D), lambda i: (i, 0)),
        out_shape=jax.ShapeDtypeStruct((N, D), jnp.float32),
    )(acc, y, degp4, b)


def kernel(x, edge_index, W1, b1, W2, b2):
    ei = edge_index.astype(jnp.int32).reshape(2 * E)
    degp = _deg(ei)[:, :N]
    degp4 = degp.reshape(NC, NG, 1, RB)
    y1 = _mmscale(x, W1, degp4)
    acc1 = _edge_scatter(y1, ei)
    y2 = _mid(acc1, y1, degp4, b1.reshape(1, D), W2)
    acc2 = _edge_scatter(y2, ei)
    return _final(acc2, y2, degp4, b2.reshape(1, D))
